# baseline (speedup 1.0000x reference)
import jax
import jax.numpy as jnp
from jax.experimental import pallas as pl

IN_CH = 128
HID = 128
NUM_NODES = 10000
WINDOW = 4
D_D = 7


def _gcn_conv(x, edge_index, edge_weight, W, b):
    N = x.shape[0]
    loop = jnp.arange(N, dtype=edge_index.dtype)
    row = jnp.concatenate([edge_index[0], loop])
    col = jnp.concatenate([edge_index[1], loop])
    ew = jnp.concatenate([edge_weight, jnp.ones((N,), x.dtype)])
    deg = jnp.zeros((N,), x.dtype).at[col].add(ew)
    dinv = jnp.where(deg > 0, 1.0 / jnp.sqrt(deg), 0.0)
    norm = dinv[row] * ew * dinv[col]
    h = x @ W
    out = jnp.zeros((N, W.shape[1]), x.dtype).at[col].add(norm[:, None] * h[row])
    return out + b


def _bn_eval(x, g, b):
    return g * (x / jnp.sqrt(1.0 + 1e-5)) + b


def _lstm(xs, Wih, Whh, bih, bhh):
    B = xs.shape[1]
    H = Whh.shape[1]
    h0 = jnp.zeros((B, H), xs.dtype)
    c0 = jnp.zeros((B, H), xs.dtype)

    def step(carry, xt):
        h, c = carry
        gates = xt @ Wih.T + h @ Whh.T + bih + bhh
        i, f, g, o = jnp.split(gates, 4, axis=1)
        i = jax.nn.sigmoid(i)
        f = jax.nn.sigmoid(f)
        g = jnp.tanh(g)
        o = jax.nn.sigmoid(o)
        c = f * c + i * g
        h = o * jnp.tanh(c)
        return (h, c), h

    (hn, cn), ys = jax.lax.scan(step, (h0, c0), xs)
    return ys, hn


def _head_kernel(h_ref, wl1_ref, bl1_ref, wl2_ref, bl2_ref, o_ref):
    h = h_ref[...]
    z = jnp.maximum(h @ wl1_ref[...].T + bl1_ref[...], 0.0)
    o_ref[...] = z @ wl2_ref[...].T + bl2_ref[...]


def kernel(x, edge_index, edge_weight, Wc1, bc1, Wc2, bc2, g1, be1, g2, be2, Wih1, Whh1, bih1, bhh1, Wih2, Whh2, bih2, bhh2, Wl1, bl1, Wl2, bl2):
    S = x.reshape(-1, WINDOW, NUM_NODES, IN_CH)
    S = jnp.transpose(S, (0, 2, 1, 3)).reshape(-1, WINDOW, IN_CH)
    parts = [S[:, 0, :]]
    for l in range(1, WINDOW):
        parts.append(S[:, l, IN_CH - 1][:, None])
    S = jnp.concatenate(parts, axis=1)
    h1 = _bn_eval(jax.nn.relu(_gcn_conv(x, edge_index, edge_weight, Wc1, bc1)), g1, be1)
    h2 = _bn_eval(jax.nn.relu(_gcn_conv(h1, edge_index, edge_weight, Wc2, bc2)), g2, be2)
    Xc = jnp.concatenate([h1, h2], axis=1)
    F2 = Xc.shape[1]
    Xc = Xc.reshape(-1, WINDOW, NUM_NODES, F2)
    Xc = jnp.transpose(Xc, (1, 0, 2, 3)).reshape(WINDOW, -1, F2)
    ys1, hn1 = _lstm(Xc, Wih1, Whh1, bih1, bhh1)
    ys2, hn2 = _lstm(ys1, Wih2, Whh2, bih2, bhh2)
    h = jnp.concatenate([hn1, hn2, S], axis=1)
    h = jax.nn.relu(h)

    BR = 400
    out = pl.pallas_call(
        _head_kernel,
        grid=(NUM_NODES // BR,),
        in_specs=[
            pl.BlockSpec((BR, h.shape[1]), lambda i: (i, 0)),
            pl.BlockSpec(Wl1.shape, lambda i: (0, 0)),
            pl.BlockSpec(bl1.shape, lambda i: (0,)),
            pl.BlockSpec(Wl2.shape, lambda i: (0, 0)),
            pl.BlockSpec(bl2.shape, lambda i: (0,)),
        ],
        out_specs=pl.BlockSpec((BR, D_D), lambda i: (i, 0)),
        out_shape=jax.ShapeDtypeStruct((NUM_NODES, D_D), jnp.float32),
    )(h, Wl1, bl1, Wl2, bl2)
    return out


# speedup vs baseline: 2.2170x; 2.2170x over previous
import functools
import math

import jax
import jax.numpy as jnp
from jax import lax
from jax.experimental import pallas as pl
from jax.experimental.pallas import tpu as pltpu
from jax.experimental.pallas import tpu_sc as plsc

IN_CH = 128
HID = 128
NUM_NODES = 10000
WINDOW = 4
D_D = 7
N_TOT = NUM_NODES * WINDOW
N_PAD = 40960
EPAD = 524288
NT = 32
NCHUNK = 4
EPT = EPAD // NT
EB = 2048
NBLOCKS = EPT // EB
SCAP = 2320
FLUSH = 2048
FINAL = 2304
CAP = 18688
CAPR = CAP // 128
GB = 128
ACC_ROWS = 10240
_BN_SCALE = 1.0 / math.sqrt(1.0 + 1e-5)

@functools.cache
def _mesh():
    return plsc.VectorSubcoreMesh(core_axis_name="c", subcore_axis_name="s")


def _bucket_body(row2, col2, ew2, bsrc, bdst, bew, counts, degp,
                 rstg, cstg, wstg,
                 ssrc0, ssrc1, ssrc2, ssrc3,
                 sdst0, sdst1, sdst2, sdst3,
                 sew0, sew1, sew2, sew3,
                 zbuf, cbuf, degsh, dsem):
    ssrc = [ssrc0, ssrc1, ssrc2, ssrc3]
    sdst = [sdst0, sdst1, sdst2, sdst3]
    sew = [sew0, sew1, sew2, sew3]
    c = lax.axis_index("c")
    s = lax.axis_index("s")
    wid = c * 16 + s
    i16 = lax.iota(jnp.int32, 16)
    z16f = jnp.zeros((16,), jnp.float32)

    def _zb(i, carry):
        zbuf[pl.ds(i * 16, 16)] = z16f
        return carry

    lax.fori_loop(0, 2560 // 16, _zb, 0)
    pltpu.sync_copy(zbuf, degsh.at[pl.ds(pl.multiple_of(s * 2560, 256), 2560)])
    plsc.subcore_barrier()

    def process_block(blk, carry):
        r0 = wid * (EPT // 128) + blk * 16
        pltpu.sync_copy(row2.at[pl.ds(r0, 16)], rstg)
        pltpu.sync_copy(col2.at[pl.ds(r0, 16)], cstg)
        pltpu.sync_copy(ew2.at[pl.ds(r0, 16)], wstg)
        for j in range(16):
            pltpu.async_copy(wstg.at[j], degsh.at[cstg.at[j]], dsem, add=True)

        def batch(b, fc):
            j = b // 8
            o = (b % 8) * 16
            rvec = rstg[j, pl.ds(o, 16)]
            cvec = cstg[j, pl.ds(o, 16)]
            wvec = wstg[j, pl.ds(o, 16)]
            fc = list(fc)
            for k in range(NCHUNK):
                fk = fc[k]
                ok = fc[NCHUNK + k]
                m = (cvec >= k * NUM_NODES) & (cvec < (k + 1) * NUM_NODES)
                mi = m.astype(jnp.int32)
                csum = mi
                for st in (1, 2, 4, 8):
                    g = csum.at[jnp.maximum(i16 - st, 0)].get(
                        mode="promise_in_bounds")
                    csum = csum + jnp.where(i16 >= st, g, 0)
                pos = fk + csum - mi
                plsc.store_scatter(ssrc[k], [pos], rvec, mask=m)
                plsc.store_scatter(sdst[k], [pos], cvec - k * NUM_NODES,
                                   mask=m)
                plsc.store_scatter(sew[k], [pos], wvec, mask=m)
                fk = fk + csum[15]
                do = fk >= FLUSH

                @pl.when(do)
                def _flush(k=k, ok=ok):
                    bb = pl.multiple_of((wid * NCHUNK + k) * CAP + ok, 256)
                    pltpu.sync_copy(ssrc[k].at[pl.ds(0, FLUSH)],
                                    bsrc.at[pl.ds(bb, FLUSH)])
                    pltpu.sync_copy(sdst[k].at[pl.ds(0, FLUSH)],
                                    bdst.at[pl.ds(bb, FLUSH)])
                    pltpu.sync_copy(sew[k].at[pl.ds(0, FLUSH)],
                                    bew.at[pl.ds(bb, FLUSH)])
                    vs = ssrc[k][pl.ds(FLUSH, 16)]
                    vd = sdst[k][pl.ds(FLUSH, 16)]
                    vw = sew[k][pl.ds(FLUSH, 16)]
                    ssrc[k][pl.ds(0, 16)] = vs
                    sdst[k][pl.ds(0, 16)] = vd
                    sew[k][pl.ds(0, 16)] = vw

                fc[k] = jnp.where(do, fk - FLUSH, fk)
                fc[NCHUNK + k] = jnp.where(do, ok + FLUSH, ok)
            return tuple(fc)

        carry = lax.fori_loop(0, EB // 16, batch, carry)
        for j in range(16):
            pltpu.make_async_copy(wstg.at[j], degsh.at[cstg.at[j]], dsem).wait()
        return carry

    carry = (0, 0, 0, 0, 0, 0, 0, 0)
    for _blk in range(NBLOCKS):
        carry = process_block(_blk, carry)

    cv = jnp.zeros((16,), jnp.int32)
    for k in range(NCHUNK):
        fk = carry[k]
        ok = carry[NCHUNK + k]
        fb = ((fk + 255) // 256) * 256
        base = jnp.maximum(fb - 256, 0)
        for m in range(16):
            p0 = base + m * 16
            keep = (p0 + i16) < fk
            ssrc[k][pl.ds(p0, 16)] = jnp.where(keep, ssrc[k][pl.ds(p0, 16)], 0)
            sdst[k][pl.ds(p0, 16)] = jnp.where(keep, sdst[k][pl.ds(p0, 16)], 0)
            sew[k][pl.ds(p0, 16)] = jnp.where(keep, sew[k][pl.ds(p0, 16)], 0.0)
        bb = pl.multiple_of((wid * NCHUNK + k) * CAP + ok, 256)
        pltpu.sync_copy(ssrc[k].at[pl.ds(0, FINAL)],
                        bsrc.at[pl.ds(bb, FINAL)])
        pltpu.sync_copy(sdst[k].at[pl.ds(0, FINAL)],
                        bdst.at[pl.ds(bb, FINAL)])
        pltpu.sync_copy(sew[k].at[pl.ds(0, FINAL)],
                        bew.at[pl.ds(bb, FINAL)])
        cv = jnp.where(i16 == k, ok + fb, cv)

    cbuf[...] = cv
    pltpu.sync_copy(cbuf, counts.at[wid])
    plsc.subcore_barrier()
    off = pl.multiple_of(s * 2560, 256)
    pltpu.sync_copy(degsh.at[pl.ds(off, 2560)],
                    degp.at[c, pl.ds(off, 2560)])


@functools.cache
def _bucket():
    return pl.kernel(
        _bucket_body,
        out_type=[
        jax.ShapeDtypeStruct((NT * NCHUNK * CAP,), jnp.int32),
        jax.ShapeDtypeStruct((NT * NCHUNK * CAP,), jnp.int32),
        jax.ShapeDtypeStruct((NT * NCHUNK * CAP,), jnp.float32),
        jax.ShapeDtypeStruct((NT, 16), jnp.int32),
        jax.ShapeDtypeStruct((2, N_PAD), jnp.float32),
    ],
        mesh=_mesh(),
        compiler_params=pltpu.CompilerParams(needs_layout_passes=False),
        scratch_types=[
        pltpu.VMEM((16, 128), jnp.int32),
        pltpu.VMEM((16, 128), jnp.int32),
        pltpu.VMEM((16, 128), jnp.float32),
        pltpu.VMEM((SCAP,), jnp.int32),
        pltpu.VMEM((SCAP,), jnp.int32),
        pltpu.VMEM((SCAP,), jnp.int32),
        pltpu.VMEM((SCAP,), jnp.int32),
        pltpu.VMEM((SCAP,), jnp.int32),
        pltpu.VMEM((SCAP,), jnp.int32),
        pltpu.VMEM((SCAP,), jnp.int32),
        pltpu.VMEM((SCAP,), jnp.int32),
        pltpu.VMEM((SCAP,), jnp.float32),
        pltpu.VMEM((SCAP,), jnp.float32),
        pltpu.VMEM((SCAP,), jnp.float32),
        pltpu.VMEM((SCAP,), jnp.float32),
        pltpu.VMEM((2560,), jnp.float32),
        pltpu.VMEM((16,), jnp.int32),
        pltpu.VMEM_SHARED((N_PAD,), jnp.float32),
        pltpu.SemaphoreType.DMA,
    ],
    )


def _spmm_body(hs, bsrc, bdst, bew, counts, agg,
               ssrc, sdst, sew, ra, rb, cbuf, acc, gA, gB, sA, sB):
    c = lax.axis_index("c")
    s = lax.axis_index("s")
    i16 = lax.iota(jnp.int32, 16)
    z16f = jnp.zeros((16,), jnp.float32)

    def _zero_ra(i, carry):
        for g in range(8):
            ra[i, pl.ds(g * 16, 16)] = z16f
        return carry

    def _scale(buf, lb):
        def srow(r, carry):
            for u in range(8):
                rr = r * 8 + u
                ewv = plsc.load_gather(
                    sew, [jnp.full((16,), lb * GB + rr, jnp.int32)])
                for g in range(8):
                    buf[rr, pl.ds(g * 16, 16)] = buf[rr, pl.ds(g * 16, 16)] * ewv
            return carry

        lax.fori_loop(0, GB // 8, srow, 0)

    def _stage(base, p):
        off = pl.multiple_of(base + p * 256, 256)
        pltpu.sync_copy(bsrc.at[pl.ds(off, 256)], ssrc)
        pltpu.sync_copy(bdst.at[pl.ds(off, 256)], sdst)
        pltpu.sync_copy(bew.at[pl.ds(off, 256)], sew)

    def _fire_gather(lb, buf, sem):
        for u in range(8):
            idx = ssrc[pl.ds(lb * GB + u * 16, 16)]
            pltpu.async_copy(hs.at[idx], buf.at[pl.ds(u * 16, 16)], sem)

    def _wait_gather(buf, sem):
        for u in range(8):
            idx = ssrc[pl.ds(u * 16, 16)]
            pltpu.make_async_copy(hs.at[idx], buf.at[pl.ds(u * 16, 16)],
                                  sem).wait()

    def _fire_scatter(lb, buf, sem):
        for u in range(8):
            idx = sdst[pl.ds(lb * GB + u * 16, 16)]
            pltpu.async_copy(buf.at[pl.ds(u * 16, 16)], acc.at[idx], sem,
                             add=True)

    def _wait_scatter(buf, sem):
        for u in range(8):
            idx = sdst[pl.ds(u * 16, 16)]
            pltpu.make_async_copy(buf.at[pl.ds(u * 16, 16)], acc.at[idx],
                                  sem).wait()

    for ci in range(2):
        k = c * 2 + ci
        plsc.subcore_barrier()
        lax.fori_loop(0, GB, _zero_ra, 0)
        for m in range(ACC_ROWS // 16 // 64):
            pltpu.sync_copy(ra.at[pl.ds(0, 64)], acc.at[pl.ds(
                pl.multiple_of(s * (ACC_ROWS // 16) + m * 64, 64), 64)])
        plsc.subcore_barrier()
        for bi in range(2):
            t = s * 2 + bi
            pltpu.sync_copy(counts.at[t], cbuf)
            cnt = cbuf[...].at[jnp.full((16,), k, jnp.int32)].get(
                mode="promise_in_bounds")[0]
            np2 = cnt // 256
            base = pl.multiple_of((t * NCHUNK + k) * CAP, 256)

            @pl.when(np2 > 0)
            def _process(k=k, np2=np2, base=base):
                _stage(base, 0)
                _fire_gather(0, ra, gA)
                _fire_gather(1, rb, gB)

                def pair(p, carry):
                    _wait_gather(ra, gA)
                    _scale(ra, 0)
                    _fire_scatter(0, ra, sA)
                    _wait_gather(rb, gB)
                    _scale(rb, 1)
                    _fire_scatter(1, rb, sB)
                    _stage(base, jnp.minimum(p + 1, np2 - 1))
                    _wait_scatter(ra, sA)
                    _fire_gather(0, ra, gA)
                    _wait_scatter(rb, sB)
                    _fire_gather(1, rb, gB)
                    return carry

                lax.fori_loop(0, np2, pair, 0)
                _wait_gather(ra, gA)
                _wait_gather(rb, gB)

        plsc.subcore_barrier()
        nr = ACC_ROWS // 16
        pltpu.sync_copy(acc.at[pl.ds(pl.multiple_of(s * nr, 64), nr)],
                        agg.at[pl.ds(pl.multiple_of(k * ACC_ROWS + s * nr, 64),
                                     nr)])


@functools.cache
def _spmm():
    return pl.kernel(
        _spmm_body,
        out_type=jax.ShapeDtypeStruct((NCHUNK * ACC_ROWS, IN_CH), jnp.float32),
        mesh=_mesh(),
        compiler_params=pltpu.CompilerParams(needs_layout_passes=False),
        scratch_types=[
        pltpu.VMEM((256,), jnp.int32),
        pltpu.VMEM((256,), jnp.int32),
        pltpu.VMEM((256,), jnp.float32),
        pltpu.VMEM((GB, 128), jnp.float32),
        pltpu.VMEM((GB, 128), jnp.float32),
        pltpu.VMEM((16,), jnp.int32),
        pltpu.VMEM_SHARED((ACC_ROWS, 128), jnp.float32),
        pltpu.SemaphoreType.DMA,
        pltpu.SemaphoreType.DMA,
        pltpu.SemaphoreType.DMA,
        pltpu.SemaphoreType.DMA,
    ],
    )


def _mm1_body(x_ref, w_ref, degp_ref, hs_ref, dinv_ref):
    d = 1.0 + degp_ref[0, :] + degp_ref[1, :]
    dinv = lax.rsqrt(d)
    h = jnp.dot(x_ref[...], w_ref[...], preferred_element_type=jnp.float32)
    hs_ref[...] = h * dinv[:, None]
    dinv_ref[...] = dinv[:, None]


def _mm1(x, w, degp):
    BR = 512
    nb = pl.cdiv(N_TOT, BR)
    return pl.pallas_call(
        _mm1_body,
        grid=(nb,),
        in_specs=[
            pl.BlockSpec((BR, IN_CH), lambda i: (i, 0)),
            pl.BlockSpec((IN_CH, HID), lambda i: (0, 0)),
            pl.BlockSpec((2, BR), lambda i: (0, i)),
        ],
        out_specs=[
            pl.BlockSpec((BR, HID), lambda i: (i, 0)),
            pl.BlockSpec((BR, 1), lambda i: (i, 0)),
        ],
        out_shape=[
            jax.ShapeDtypeStruct((N_TOT, HID), jnp.float32),
            jax.ShapeDtypeStruct((N_TOT, 1), jnp.float32),
        ],
    )(x, w, degp)


def _mm2_body(agg_ref, hs_ref, dinv_ref, bc_ref, g_ref, be_ref, w_ref,
              h1_ref, hs2_ref):
    dinv = dinv_ref[...]
    pre = dinv * (agg_ref[...] + hs_ref[...]) + bc_ref[...]
    h1 = jnp.maximum(pre, 0.0)
    h1 = g_ref[...] * (h1 * _BN_SCALE) + be_ref[...]
    h1_ref[...] = h1
    hs2_ref[...] = jnp.dot(
        h1, w_ref[...], preferred_element_type=jnp.float32) * dinv


def _mm2(agg, hs, dinv, bc, g, be, w):
    BR = 512
    nb = pl.cdiv(N_TOT, BR)
    return pl.pallas_call(
        _mm2_body,
        grid=(nb,),
        in_specs=[
            pl.BlockSpec((BR, HID), lambda i: (i, 0)),
            pl.BlockSpec((BR, HID), lambda i: (i, 0)),
            pl.BlockSpec((BR, 1), lambda i: (i, 0)),
            pl.BlockSpec((1, HID), lambda i: (0, 0)),
            pl.BlockSpec((1, HID), lambda i: (0, 0)),
            pl.BlockSpec((1, HID), lambda i: (0, 0)),
            pl.BlockSpec((HID, HID), lambda i: (0, 0)),
        ],
        out_specs=[
            pl.BlockSpec((BR, HID), lambda i: (i, 0)),
            pl.BlockSpec((BR, HID), lambda i: (i, 0)),
        ],
        out_shape=[
            jax.ShapeDtypeStruct((N_TOT, HID), jnp.float32),
            jax.ShapeDtypeStruct((N_TOT, HID), jnp.float32),
        ],
    )(agg, hs, dinv, bc, g, be, w)


def _sigm(v):
    return jax.nn.sigmoid(v)


def _lstm_head_body(h1_ref, agg2_ref, hs2_ref, dinv_ref, bc2_ref, g2_ref,
                    be2_ref, wih1_ref, whh1_ref, b1_ref, wih2_ref, whh2_ref,
                    b2_ref, x_ref, wl1_ref, bl1_ref, wl2_ref, bl2_ref, o_ref):
    BR = h1_ref.shape[1]
    h1 = h1_ref[...]
    dinv = dinv_ref[...]
    h2 = dinv * (agg2_ref[...] + hs2_ref[...]) + bc2_ref[...]
    h2 = jnp.maximum(h2, 0.0)
    h2 = g2_ref[...] * (h2 * _BN_SCALE) + be2_ref[...]
    wih1 = wih1_ref[...]
    whh1 = whh1_ref[...]
    b1 = b1_ref[...]
    wih2 = wih2_ref[...]
    whh2 = whh2_ref[...]
    b2 = b2_ref[...]
    hA = jnp.zeros((BR, HID), jnp.float32)
    cA = jnp.zeros((BR, HID), jnp.float32)
    hB = jnp.zeros((BR, HID), jnp.float32)
    cB = jnp.zeros((BR, HID), jnp.float32)
    for t in range(WINDOW):
        xt = jnp.concatenate([h1[t], h2[t]], axis=1)
        gates = (jnp.dot(xt, wih1.T, preferred_element_type=jnp.float32)
                 + jnp.dot(hA, whh1.T, preferred_element_type=jnp.float32)
                 + b1)
        ig, fg, gg, og = jnp.split(gates, 4, axis=1)
        cA = _sigm(fg) * cA + _sigm(ig) * jnp.tanh(gg)
        hA = _sigm(og) * jnp.tanh(cA)
        gates = (jnp.dot(hA, wih2.T, preferred_element_type=jnp.float32)
                 + jnp.dot(hB, whh2.T, preferred_element_type=jnp.float32)
                 + b2)
        ig, fg, gg, og = jnp.split(gates, 4, axis=1)
        cB = _sigm(fg) * cB + _sigm(ig) * jnp.tanh(gg)
        hB = _sigm(og) * jnp.tanh(cB)
    x4 = x_ref[...]
    S = jnp.concatenate(
        [x4[0], x4[1][:, IN_CH - 1:], x4[2][:, IN_CH - 1:],
         x4[3][:, IN_CH - 1:]], axis=1)
    hcat = jnp.maximum(jnp.concatenate([hA, hB, S], axis=1), 0.0)
    z = jnp.maximum(
        jnp.dot(hcat, wl1_ref[...].T, preferred_element_type=jnp.float32)
        + bl1_ref[...], 0.0)
    o_ref[...] = (jnp.dot(z, wl2_ref[...].T,
                          preferred_element_type=jnp.float32) + bl2_ref[...])


def _lstm_head(h1r, agg2r, hs2r, dinvr, bc2, g2, be2, Wih1, Whh1, b1,
               Wih2, Whh2, b2, xr, Wl1, bl1, Wl2, bl2):
    BR = 512
    nb = pl.cdiv(NUM_NODES, BR)
    d1 = 2 * HID + IN_CH + WINDOW - 1
    full = lambda shape: pl.BlockSpec(shape, lambda i: tuple(0 for _ in shape))
    return pl.pallas_call(
        _lstm_head_body,
        grid=(nb,),
        in_specs=[
            pl.BlockSpec((WINDOW, BR, HID), lambda i: (0, i, 0)),
            pl.BlockSpec((WINDOW, BR, HID), lambda i: (0, i, 0)),
            pl.BlockSpec((WINDOW, BR, HID), lambda i: (0, i, 0)),
            pl.BlockSpec((WINDOW, BR, 1), lambda i: (0, i, 0)),
            full((1, HID)),
            full((1, HID)),
            full((1, HID)),
            full((4 * HID, 2 * HID)),
            full((4 * HID, HID)),
            full((1, 4 * HID)),
            full((4 * HID, HID)),
            full((4 * HID, HID)),
            full((1, 4 * HID)),
            pl.BlockSpec((WINDOW, BR, IN_CH), lambda i: (0, i, 0)),
            full((HID, d1)),
            full((1, HID)),
            full((D_D, HID)),
            full((1, D_D)),
        ],
        out_specs=pl.BlockSpec((BR, D_D), lambda i: (i, 0)),
        out_shape=jax.ShapeDtypeStruct((NUM_NODES, D_D), jnp.float32),
    )(h1r, agg2r, hs2r, dinvr, bc2, g2, be2, Wih1, Whh1, b1, Wih2, Whh2, b2,
      xr, Wl1, bl1, Wl2, bl2)


def kernel(x, edge_index, edge_weight, Wc1, bc1, Wc2, bc2, g1, be1, g2, be2,
           Wih1, Whh1, bih1, bhh1, Wih2, Whh2, bih2, bhh2, Wl1, bl1, Wl2, bl2):
    e = edge_weight.shape[0]
    pad = EPAD - e
    row2 = jnp.concatenate(
        [edge_index[0], jnp.zeros((pad,), edge_index.dtype)]).reshape(-1, 128)
    col2 = jnp.concatenate(
        [edge_index[1], jnp.zeros((pad,), edge_index.dtype)]).reshape(-1, 128)
    ew2 = jnp.concatenate(
        [edge_weight, jnp.zeros((pad,), edge_weight.dtype)]).reshape(-1, 128)

    bsrc, bdst, bew, counts, degp = _bucket()(row2, col2, ew2)

    unpad = lambda a: a.reshape(NCHUNK, ACC_ROWS, IN_CH)[:, :NUM_NODES, :]
    hs1, dinv = _mm1(x, Wc1, degp)
    agg1 = unpad(_spmm()(hs1, bsrc, bdst, bew, counts)).reshape(N_TOT, IN_CH)
    h1, hs2 = _mm2(agg1, hs1, dinv, bc1.reshape(1, -1), g1.reshape(1, -1),
                   be1.reshape(1, -1), Wc2)
    agg2 = unpad(_spmm()(hs2, bsrc, bdst, bew, counts)).reshape(N_TOT, IN_CH)

    r4 = lambda a: a.reshape(WINDOW, NUM_NODES, -1)
    out = _lstm_head(
        r4(h1), r4(agg2), r4(hs2), r4(dinv), bc2.reshape(1, -1),
        g2.reshape(1, -1), be2.reshape(1, -1), Wih1, Whh1,
        (bih1 + bhh1).reshape(1, -1), Wih2, Whh2,
        (bih2 + bhh2).reshape(1, -1), r4(x), Wl1, bl1.reshape(1, -1),
        Wl2, bl2.reshape(1, -1))
    return out


# speedup vs baseline: 9.1747x; 4.1383x over previous
import functools
import math

import jax
import jax.numpy as jnp
from jax import lax
from jax.experimental import pallas as pl
from jax.experimental.pallas import tpu as pltpu
from jax.experimental.pallas import tpu_sc as plsc

IN_CH = 128
HID = 128
NUM_NODES = 10000
WINDOW = 4
D_D = 7
N_TOT = NUM_NODES * WINDOW
N_PAD = 40960
EPAD = 524288
NT = 32
NCHUNK = 4
EPT = EPAD // NT
EB = 2048
NBLOCKS = EPT // EB
SCAP = 2320
FLUSH = 2048
FINAL = 2304
CAP = 18688
CAPR = CAP // 128
GB = 128
ACC_ROWS = 10240
_BN_SCALE = 1.0 / math.sqrt(1.0 + 1e-5)

@functools.cache
def _mesh():
    return plsc.VectorSubcoreMesh(core_axis_name="c", subcore_axis_name="s")


def _bucket_body(row2, col2, ew2, bsrc, bdst, bew, counts, degp,
                 rstg, cstg, wstg,
                 ssrc0, ssrc1, ssrc2, ssrc3,
                 sdst0, sdst1, sdst2, sdst3,
                 sew0, sew1, sew2, sew3,
                 zbuf, cbuf, degsh, dsem):
    ssrc = [ssrc0, ssrc1, ssrc2, ssrc3]
    sdst = [sdst0, sdst1, sdst2, sdst3]
    sew = [sew0, sew1, sew2, sew3]
    c = lax.axis_index("c")
    s = lax.axis_index("s")
    wid = c * 16 + s
    i16 = lax.iota(jnp.int32, 16)
    z16f = jnp.zeros((16,), jnp.float32)

    def _zb(i, carry):
        zbuf[pl.ds(i * 16, 16)] = z16f
        return carry

    lax.fori_loop(0, 2560 // 16, _zb, 0)
    pltpu.sync_copy(zbuf, degsh.at[pl.ds(pl.multiple_of(s * 2560, 256), 2560)])
    plsc.subcore_barrier()

    def process_block(blk, carry):
        r0 = wid * (EPT // 128) + blk * 16
        pltpu.sync_copy(row2.at[pl.ds(r0, 16)], rstg)
        pltpu.sync_copy(col2.at[pl.ds(r0, 16)], cstg)
        pltpu.sync_copy(ew2.at[pl.ds(r0, 16)], wstg)
        for j in range(16):
            pltpu.async_copy(wstg.at[j], degsh.at[cstg.at[j]], dsem, add=True)

        def batch(b, fc):
            j = b // 8
            o = (b % 8) * 16
            rvec = rstg[j, pl.ds(o, 16)]
            cvec = cstg[j, pl.ds(o, 16)]
            wvec = wstg[j, pl.ds(o, 16)]
            fc = list(fc)
            for k in range(NCHUNK):
                fk = fc[k]
                ok = fc[NCHUNK + k]
                m = (cvec >= k * NUM_NODES) & (cvec < (k + 1) * NUM_NODES)
                mi = m.astype(jnp.int32)
                csum = mi
                for st in (1, 2, 4, 8):
                    g = csum.at[jnp.maximum(i16 - st, 0)].get(
                        mode="promise_in_bounds")
                    csum = csum + jnp.where(i16 >= st, g, 0)
                pos = fk + csum - mi
                plsc.store_scatter(ssrc[k], [pos], rvec, mask=m)
                plsc.store_scatter(sdst[k], [pos], cvec - k * NUM_NODES,
                                   mask=m)
                plsc.store_scatter(sew[k], [pos], wvec, mask=m)
                fk = fk + csum[15]
                do = fk >= FLUSH

                @pl.when(do)
                def _flush(k=k, ok=ok):
                    bb = pl.multiple_of((wid * NCHUNK + k) * CAP + ok, 256)
                    pltpu.sync_copy(ssrc[k].at[pl.ds(0, FLUSH)],
                                    bsrc.at[pl.ds(bb, FLUSH)])
                    pltpu.sync_copy(sdst[k].at[pl.ds(0, FLUSH)],
                                    bdst.at[pl.ds(bb, FLUSH)])
                    pltpu.sync_copy(sew[k].at[pl.ds(0, FLUSH)],
                                    bew.at[pl.ds(bb, FLUSH)])
                    vs = ssrc[k][pl.ds(FLUSH, 16)]
                    vd = sdst[k][pl.ds(FLUSH, 16)]
                    vw = sew[k][pl.ds(FLUSH, 16)]
                    ssrc[k][pl.ds(0, 16)] = vs
                    sdst[k][pl.ds(0, 16)] = vd
                    sew[k][pl.ds(0, 16)] = vw

                fc[k] = jnp.where(do, fk - FLUSH, fk)
                fc[NCHUNK + k] = jnp.where(do, ok + FLUSH, ok)
            return tuple(fc)

        carry = lax.fori_loop(0, EB // 16, batch, carry)
        for j in range(16):
            pltpu.make_async_copy(wstg.at[j], degsh.at[cstg.at[j]], dsem).wait()
        return carry

    carry = (0, 0, 0, 0, 0, 0, 0, 0)
    for _blk in range(NBLOCKS):
        carry = process_block(_blk, carry)

    cv = jnp.zeros((16,), jnp.int32)
    for k in range(NCHUNK):
        fk = carry[k]
        ok = carry[NCHUNK + k]
        fb = ((fk + 255) // 256) * 256
        base = jnp.maximum(fb - 256, 0)
        for m in range(16):
            p0 = base + m * 16
            pos = p0 + i16
            keep = pos < fk
            sprd = (wid * 289 + pos * 37) & 8191
            ssrc[k][pl.ds(p0, 16)] = jnp.where(keep, ssrc[k][pl.ds(p0, 16)],
                                               sprd)
            sdst[k][pl.ds(p0, 16)] = jnp.where(keep, sdst[k][pl.ds(p0, 16)],
                                               sprd)
            sew[k][pl.ds(p0, 16)] = jnp.where(keep, sew[k][pl.ds(p0, 16)], 0.0)
        bb = pl.multiple_of((wid * NCHUNK + k) * CAP + ok, 256)
        pltpu.sync_copy(ssrc[k].at[pl.ds(0, FINAL)],
                        bsrc.at[pl.ds(bb, FINAL)])
        pltpu.sync_copy(sdst[k].at[pl.ds(0, FINAL)],
                        bdst.at[pl.ds(bb, FINAL)])
        pltpu.sync_copy(sew[k].at[pl.ds(0, FINAL)],
                        bew.at[pl.ds(bb, FINAL)])
        cv = jnp.where(i16 == k, ok + fb, cv)

    cbuf[...] = cv
    pltpu.sync_copy(cbuf, counts.at[wid])
    plsc.subcore_barrier()
    off = pl.multiple_of(s * 2560, 256)
    pltpu.sync_copy(degsh.at[pl.ds(off, 2560)],
                    degp.at[c, pl.ds(off, 2560)])


@functools.cache
def _bucket():
    return pl.kernel(
        _bucket_body,
        out_type=[
        jax.ShapeDtypeStruct((NT * NCHUNK * CAP,), jnp.int32),
        jax.ShapeDtypeStruct((NT * NCHUNK * CAP,), jnp.int32),
        jax.ShapeDtypeStruct((NT * NCHUNK * CAP,), jnp.float32),
        jax.ShapeDtypeStruct((NT, 16), jnp.int32),
        jax.ShapeDtypeStruct((2, N_PAD), jnp.float32),
    ],
        mesh=_mesh(),
        compiler_params=pltpu.CompilerParams(needs_layout_passes=False),
        scratch_types=[
        pltpu.VMEM((16, 128), jnp.int32),
        pltpu.VMEM((16, 128), jnp.int32),
        pltpu.VMEM((16, 128), jnp.float32),
        pltpu.VMEM((SCAP,), jnp.int32),
        pltpu.VMEM((SCAP,), jnp.int32),
        pltpu.VMEM((SCAP,), jnp.int32),
        pltpu.VMEM((SCAP,), jnp.int32),
        pltpu.VMEM((SCAP,), jnp.int32),
        pltpu.VMEM((SCAP,), jnp.int32),
        pltpu.VMEM((SCAP,), jnp.int32),
        pltpu.VMEM((SCAP,), jnp.int32),
        pltpu.VMEM((SCAP,), jnp.float32),
        pltpu.VMEM((SCAP,), jnp.float32),
        pltpu.VMEM((SCAP,), jnp.float32),
        pltpu.VMEM((SCAP,), jnp.float32),
        pltpu.VMEM((2560,), jnp.float32),
        pltpu.VMEM((16,), jnp.int32),
        pltpu.VMEM_SHARED((N_PAD,), jnp.float32),
        pltpu.SemaphoreType.DMA,
    ],
    )


def _spmm_body(hs, bsrc, bdst, bew, counts, agg,
               ssrc, sdst, sew, ra, rb, cbuf, acc, gA, gB, sA, sB):
    c = lax.axis_index("c")
    s = lax.axis_index("s")
    i16 = lax.iota(jnp.int32, 16)
    z16f = jnp.zeros((16,), jnp.float32)

    def _zero_ra(i, carry):
        for g in range(8):
            ra[i, pl.ds(g * 16, 16)] = z16f
        return carry

    def _scale(buf, lb):
        def srow(r, carry):
            for u in range(8):
                rr = r * 8 + u
                ewv = plsc.load_gather(
                    sew, [jnp.full((16,), lb * GB + rr, jnp.int32)])
                for g in range(8):
                    buf[rr, pl.ds(g * 16, 16)] = buf[rr, pl.ds(g * 16, 16)] * ewv
            return carry

        lax.fori_loop(0, GB // 8, srow, 0)

    def _stage(base, p):
        off = pl.multiple_of(base + p * 256, 256)
        pltpu.sync_copy(bsrc.at[pl.ds(off, 256)], ssrc)
        pltpu.sync_copy(bdst.at[pl.ds(off, 256)], sdst)
        pltpu.sync_copy(bew.at[pl.ds(off, 256)], sew)

    def _fire_gather(lb, buf, sem):
        for u in range(8):
            idx = ssrc[pl.ds(lb * GB + u * 16, 16)]
            pltpu.async_copy(hs.at[idx], buf.at[pl.ds(u * 16, 16)], sem)

    def _wait_gather(buf, sem):
        for u in range(8):
            idx = ssrc[pl.ds(u * 16, 16)]
            pltpu.make_async_copy(hs.at[idx], buf.at[pl.ds(u * 16, 16)],
                                  sem).wait()

    def _fire_scatter(lb, buf, sem):
        for u in range(8):
            idx = sdst[pl.ds(lb * GB + u * 16, 16)]
            pltpu.async_copy(buf.at[pl.ds(u * 16, 16)], acc.at[idx], sem,
                             add=True)

    def _wait_scatter(buf, sem):
        for u in range(8):
            idx = sdst[pl.ds(u * 16, 16)]
            pltpu.make_async_copy(buf.at[pl.ds(u * 16, 16)], acc.at[idx],
                                  sem).wait()

    for ci in range(2):
        k = c * 2 + ci
        plsc.subcore_barrier()
        lax.fori_loop(0, GB, _zero_ra, 0)
        for m in range(ACC_ROWS // 16 // 64):
            pltpu.sync_copy(ra.at[pl.ds(0, 64)], acc.at[pl.ds(
                pl.multiple_of(s * (ACC_ROWS // 16) + m * 64, 64), 64)])
        plsc.subcore_barrier()
        for bi in range(2):
            t = s * 2 + bi
            pltpu.sync_copy(counts.at[t], cbuf)
            cnt = cbuf[...].at[jnp.full((16,), k, jnp.int32)].get(
                mode="promise_in_bounds")[0]
            np2 = cnt // 256
            base = pl.multiple_of((t * NCHUNK + k) * CAP, 256)

            @pl.when(np2 > 0)
            def _process(k=k, np2=np2, base=base):
                _stage(base, 0)
                _fire_gather(0, ra, gA)
                _fire_gather(1, rb, gB)

                def pair(p, carry):
                    _wait_gather(ra, gA)
                    _scale(ra, 0)
                    _fire_scatter(0, ra, sA)
                    _wait_gather(rb, gB)
                    _scale(rb, 1)
                    _fire_scatter(1, rb, sB)
                    _stage(base, jnp.minimum(p + 1, np2 - 1))
                    _wait_scatter(ra, sA)
                    _fire_gather(0, ra, gA)
                    _wait_scatter(rb, sB)
                    _fire_gather(1, rb, gB)
                    return carry

                lax.fori_loop(0, np2, pair, 0)
                _wait_gather(ra, gA)
                _wait_gather(rb, gB)

        plsc.subcore_barrier()
        nr = ACC_ROWS // 16
        pltpu.sync_copy(acc.at[pl.ds(pl.multiple_of(s * nr, 64), nr)],
                        agg.at[pl.ds(pl.multiple_of(k * ACC_ROWS + s * nr, 64),
                                     nr)])


@functools.cache
def _spmm():
    return pl.kernel(
        _spmm_body,
        out_type=jax.ShapeDtypeStruct((NCHUNK * ACC_ROWS, IN_CH), jnp.float32),
        mesh=_mesh(),
        compiler_params=pltpu.CompilerParams(needs_layout_passes=False),
        scratch_types=[
        pltpu.VMEM((256,), jnp.int32),
        pltpu.VMEM((256,), jnp.int32),
        pltpu.VMEM((256,), jnp.float32),
        pltpu.VMEM((GB, 128), jnp.float32),
        pltpu.VMEM((GB, 128), jnp.float32),
        pltpu.VMEM((16,), jnp.int32),
        pltpu.VMEM_SHARED((ACC_ROWS, 128), jnp.float32),
        pltpu.SemaphoreType.DMA,
        pltpu.SemaphoreType.DMA,
        pltpu.SemaphoreType.DMA,
        pltpu.SemaphoreType.DMA,
    ],
    )


def _mm1_body(x_ref, w_ref, degp_ref, hs_ref, dinv_ref):
    d = 1.0 + degp_ref[0, :] + degp_ref[1, :]
    dinv = lax.rsqrt(d)
    h = jnp.dot(x_ref[...], w_ref[...], preferred_element_type=jnp.float32)
    hs_ref[...] = h * dinv[:, None]
    dinv_ref[...] = dinv[:, None]


def _mm1(x, w, degp):
    BR = 512
    nb = pl.cdiv(N_TOT, BR)
    return pl.pallas_call(
        _mm1_body,
        grid=(nb,),
        in_specs=[
            pl.BlockSpec((BR, IN_CH), lambda i: (i, 0)),
            pl.BlockSpec((IN_CH, HID), lambda i: (0, 0)),
            pl.BlockSpec((2, BR), lambda i: (0, i)),
        ],
        out_specs=[
            pl.BlockSpec((BR, HID), lambda i: (i, 0)),
            pl.BlockSpec((BR, 1), lambda i: (i, 0)),
        ],
        out_shape=[
            jax.ShapeDtypeStruct((N_TOT, HID), jnp.float32),
            jax.ShapeDtypeStruct((N_TOT, 1), jnp.float32),
        ],
    )(x, w, degp)


def _mm2_body(agg_ref, hs_ref, dinv_ref, bc_ref, g_ref, be_ref, w_ref,
              h1_ref, hs2_ref):
    dinv = dinv_ref[...]
    pre = dinv * (agg_ref[...] + hs_ref[...]) + bc_ref[...]
    h1 = jnp.maximum(pre, 0.0)
    h1 = g_ref[...] * (h1 * _BN_SCALE) + be_ref[...]
    h1_ref[...] = h1
    hs2_ref[...] = jnp.dot(
        h1, w_ref[...], preferred_element_type=jnp.float32) * dinv


def _mm2(agg, hs, dinv, bc, g, be, w):
    BR = 512
    nb = pl.cdiv(N_TOT, BR)
    return pl.pallas_call(
        _mm2_body,
        grid=(nb,),
        in_specs=[
            pl.BlockSpec((BR, HID), lambda i: (i, 0)),
            pl.BlockSpec((BR, HID), lambda i: (i, 0)),
            pl.BlockSpec((BR, 1), lambda i: (i, 0)),
            pl.BlockSpec((1, HID), lambda i: (0, 0)),
            pl.BlockSpec((1, HID), lambda i: (0, 0)),
            pl.BlockSpec((1, HID), lambda i: (0, 0)),
            pl.BlockSpec((HID, HID), lambda i: (0, 0)),
        ],
        out_specs=[
            pl.BlockSpec((BR, HID), lambda i: (i, 0)),
            pl.BlockSpec((BR, HID), lambda i: (i, 0)),
        ],
        out_shape=[
            jax.ShapeDtypeStruct((N_TOT, HID), jnp.float32),
            jax.ShapeDtypeStruct((N_TOT, HID), jnp.float32),
        ],
    )(agg, hs, dinv, bc, g, be, w)


def _sigm(v):
    return jax.nn.sigmoid(v)


def _lstm_head_body(h1_ref, agg2_ref, hs2_ref, dinv_ref, bc2_ref, g2_ref,
                    be2_ref, wih1_ref, whh1_ref, b1_ref, wih2_ref, whh2_ref,
                    b2_ref, x_ref, wl1_ref, bl1_ref, wl2_ref, bl2_ref, o_ref):
    BR = h1_ref.shape[1]
    h1 = h1_ref[...]
    dinv = dinv_ref[...]
    h2 = dinv * (agg2_ref[...] + hs2_ref[...]) + bc2_ref[...]
    h2 = jnp.maximum(h2, 0.0)
    h2 = g2_ref[...] * (h2 * _BN_SCALE) + be2_ref[...]
    wih1 = wih1_ref[...]
    whh1 = whh1_ref[...]
    b1 = b1_ref[...]
    wih2 = wih2_ref[...]
    whh2 = whh2_ref[...]
    b2 = b2_ref[...]
    hA = jnp.zeros((BR, HID), jnp.float32)
    cA = jnp.zeros((BR, HID), jnp.float32)
    hB = jnp.zeros((BR, HID), jnp.float32)
    cB = jnp.zeros((BR, HID), jnp.float32)
    for t in range(WINDOW):
        xt = jnp.concatenate([h1[t], h2[t]], axis=1)
        gates = (jnp.dot(xt, wih1.T, preferred_element_type=jnp.float32)
                 + jnp.dot(hA, whh1.T, preferred_element_type=jnp.float32)
                 + b1)
        ig, fg, gg, og = jnp.split(gates, 4, axis=1)
        cA = _sigm(fg) * cA + _sigm(ig) * jnp.tanh(gg)
        hA = _sigm(og) * jnp.tanh(cA)
        gates = (jnp.dot(hA, wih2.T, preferred_element_type=jnp.float32)
                 + jnp.dot(hB, whh2.T, preferred_element_type=jnp.float32)
                 + b2)
        ig, fg, gg, og = jnp.split(gates, 4, axis=1)
        cB = _sigm(fg) * cB + _sigm(ig) * jnp.tanh(gg)
        hB = _sigm(og) * jnp.tanh(cB)
    x4 = x_ref[...]
    S = jnp.concatenate(
        [x4[0], x4[1][:, IN_CH - 1:], x4[2][:, IN_CH - 1:],
         x4[3][:, IN_CH - 1:]], axis=1)
    hcat = jnp.maximum(jnp.concatenate([hA, hB, S], axis=1), 0.0)
    z = jnp.maximum(
        jnp.dot(hcat, wl1_ref[...].T, preferred_element_type=jnp.float32)
        + bl1_ref[...], 0.0)
    o_ref[...] = (jnp.dot(z, wl2_ref[...].T,
                          preferred_element_type=jnp.float32) + bl2_ref[...])


def _lstm_head(h1r, agg2r, hs2r, dinvr, bc2, g2, be2, Wih1, Whh1, b1,
               Wih2, Whh2, b2, xr, Wl1, bl1, Wl2, bl2):
    BR = 512
    nb = pl.cdiv(NUM_NODES, BR)
    d1 = 2 * HID + IN_CH + WINDOW - 1
    full = lambda shape: pl.BlockSpec(shape, lambda i: tuple(0 for _ in shape))
    return pl.pallas_call(
        _lstm_head_body,
        grid=(nb,),
        in_specs=[
            pl.BlockSpec((WINDOW, BR, HID), lambda i: (0, i, 0)),
            pl.BlockSpec((WINDOW, BR, HID), lambda i: (0, i, 0)),
            pl.BlockSpec((WINDOW, BR, HID), lambda i: (0, i, 0)),
            pl.BlockSpec((WINDOW, BR, 1), lambda i: (0, i, 0)),
            full((1, HID)),
            full((1, HID)),
            full((1, HID)),
            full((4 * HID, 2 * HID)),
            full((4 * HID, HID)),
            full((1, 4 * HID)),
            full((4 * HID, HID)),
            full((4 * HID, HID)),
            full((1, 4 * HID)),
            pl.BlockSpec((WINDOW, BR, IN_CH), lambda i: (0, i, 0)),
            full((HID, d1)),
            full((1, HID)),
            full((D_D, HID)),
            full((1, D_D)),
        ],
        out_specs=pl.BlockSpec((BR, D_D), lambda i: (i, 0)),
        out_shape=jax.ShapeDtypeStruct((NUM_NODES, D_D), jnp.float32),
    )(h1r, agg2r, hs2r, dinvr, bc2, g2, be2, Wih1, Whh1, b1, Wih2, Whh2, b2,
      xr, Wl1, bl1, Wl2, bl2)


def kernel(x, edge_index, edge_weight, Wc1, bc1, Wc2, bc2, g1, be1, g2, be2,
           Wih1, Whh1, bih1, bhh1, Wih2, Whh2, bih2, bhh2, Wl1, bl1, Wl2, bl2):
    e = edge_weight.shape[0]
    pad = EPAD - e
    spread = jnp.arange(pad, dtype=edge_index.dtype) % N_TOT
    row2 = jnp.concatenate([edge_index[0], spread]).reshape(-1, 128)
    col2 = jnp.concatenate([edge_index[1], spread]).reshape(-1, 128)
    ew2 = jnp.concatenate(
        [edge_weight, jnp.zeros((pad,), edge_weight.dtype)]).reshape(-1, 128)

    bsrc, bdst, bew, counts, degp = _bucket()(row2, col2, ew2)

    unpad = lambda a: a.reshape(NCHUNK, ACC_ROWS, IN_CH)[:, :NUM_NODES, :]
    hs1, dinv = _mm1(x, Wc1, degp)
    agg1 = unpad(_spmm()(hs1, bsrc, bdst, bew, counts)).reshape(N_TOT, IN_CH)
    h1, hs2 = _mm2(agg1, hs1, dinv, bc1.reshape(1, -1), g1.reshape(1, -1),
                   be1.reshape(1, -1), Wc2)
    agg2 = unpad(_spmm()(hs2, bsrc, bdst, bew, counts)).reshape(N_TOT, IN_CH)

    r4 = lambda a: a.reshape(WINDOW, NUM_NODES, -1)
    out = _lstm_head(
        r4(h1), r4(agg2), r4(hs2), r4(dinv), bc2.reshape(1, -1),
        g2.reshape(1, -1), be2.reshape(1, -1), Wih1, Whh1,
        (bih1 + bhh1).reshape(1, -1), Wih2, Whh2,
        (bih2 + bhh2).reshape(1, -1), r4(x), Wl1, bl1.reshape(1, -1),
        Wl2, bl2.reshape(1, -1))
    return out


# speedup vs baseline: 11.3471x; 1.2368x over previous
import functools
import math

import jax
import jax.numpy as jnp
from jax import lax
from jax.experimental import pallas as pl
from jax.experimental.pallas import tpu as pltpu
from jax.experimental.pallas import tpu_sc as plsc

IN_CH = 128
HID = 128
NUM_NODES = 10000
WINDOW = 4
D_D = 7
N_TOT = NUM_NODES * WINDOW
N_PAD = 40960
EPAD = 524288
NT = 32
NCHUNK = 4
EPT = EPAD // NT
EB = 2048
NBLOCKS = EPT // EB
SCAP = 2320
FLUSH = 2048
FINAL = 2304
CAP = 18688
CAPR = CAP // 128
GB = 128
ACC_ROWS = 10240
_BN_SCALE = 1.0 / math.sqrt(1.0 + 1e-5)

@functools.cache
def _mesh():
    return plsc.VectorSubcoreMesh(core_axis_name="c", subcore_axis_name="s")


def _bucket_body(row2, col2, ew2, bsrc, bdst, bew, counts, degp,
                 rstg, cstg, wstg,
                 ssrc0, ssrc1, ssrc2, ssrc3,
                 sdst0, sdst1, sdst2, sdst3,
                 sew0, sew1, sew2, sew3,
                 zbuf, cbuf, degsh, dsem):
    ssrc = [ssrc0, ssrc1, ssrc2, ssrc3]
    sdst = [sdst0, sdst1, sdst2, sdst3]
    sew = [sew0, sew1, sew2, sew3]
    c = lax.axis_index("c")
    s = lax.axis_index("s")
    wid = c * 16 + s
    i16 = lax.iota(jnp.int32, 16)
    z16f = jnp.zeros((16,), jnp.float32)

    def _zb(i, carry):
        zbuf[pl.ds(i * 16, 16)] = z16f
        return carry

    lax.fori_loop(0, 2560 // 16, _zb, 0)
    pltpu.sync_copy(zbuf, degsh.at[pl.ds(pl.multiple_of(s * 2560, 256), 2560)])
    plsc.subcore_barrier()

    def process_block(blk, carry):
        r0 = wid * (EPT // 128) + blk * 16
        pltpu.sync_copy(row2.at[pl.ds(r0, 16)], rstg)
        pltpu.sync_copy(col2.at[pl.ds(r0, 16)], cstg)
        pltpu.sync_copy(ew2.at[pl.ds(r0, 16)], wstg)
        for j in range(16):
            pltpu.async_copy(wstg.at[j], degsh.at[cstg.at[j]], dsem, add=True)

        def batch(b, fc):
            j = b // 8
            o = (b % 8) * 16
            rvec = rstg[j, pl.ds(o, 16)]
            cvec = cstg[j, pl.ds(o, 16)]
            wvec = wstg[j, pl.ds(o, 16)]
            fc = list(fc)
            for k in range(NCHUNK):
                fk = fc[k]
                ok = fc[NCHUNK + k]
                m = (cvec >= k * NUM_NODES) & (cvec < (k + 1) * NUM_NODES)
                mi = m.astype(jnp.int32)
                csum = mi
                for st in (1, 2, 4, 8):
                    g = csum.at[jnp.maximum(i16 - st, 0)].get(
                        mode="promise_in_bounds")
                    csum = csum + jnp.where(i16 >= st, g, 0)
                pos = fk + csum - mi
                plsc.store_scatter(ssrc[k], [pos], rvec, mask=m)
                plsc.store_scatter(sdst[k], [pos], cvec - k * NUM_NODES,
                                   mask=m)
                plsc.store_scatter(sew[k], [pos], wvec, mask=m)
                fk = fk + csum[15]
                do = fk >= FLUSH

                @pl.when(do)
                def _flush(k=k, ok=ok):
                    bb = pl.multiple_of((wid * NCHUNK + k) * CAP + ok, 256)
                    pltpu.sync_copy(ssrc[k].at[pl.ds(0, FLUSH)],
                                    bsrc.at[pl.ds(bb, FLUSH)])
                    pltpu.sync_copy(sdst[k].at[pl.ds(0, FLUSH)],
                                    bdst.at[pl.ds(bb, FLUSH)])
                    pltpu.sync_copy(sew[k].at[pl.ds(0, FLUSH)],
                                    bew.at[pl.ds(bb, FLUSH)])
                    vs = ssrc[k][pl.ds(FLUSH, 16)]
                    vd = sdst[k][pl.ds(FLUSH, 16)]
                    vw = sew[k][pl.ds(FLUSH, 16)]
                    ssrc[k][pl.ds(0, 16)] = vs
                    sdst[k][pl.ds(0, 16)] = vd
                    sew[k][pl.ds(0, 16)] = vw

                fc[k] = jnp.where(do, fk - FLUSH, fk)
                fc[NCHUNK + k] = jnp.where(do, ok + FLUSH, ok)
            return tuple(fc)

        carry = lax.fori_loop(0, EB // 16, batch, carry)
        for j in range(16):
            pltpu.make_async_copy(wstg.at[j], degsh.at[cstg.at[j]], dsem).wait()
        return carry

    carry = (0, 0, 0, 0, 0, 0, 0, 0)
    for _blk in range(NBLOCKS):
        carry = process_block(_blk, carry)

    cv = jnp.zeros((16,), jnp.int32)
    for k in range(NCHUNK):
        fk = carry[k]
        ok = carry[NCHUNK + k]
        fb = ((fk + 255) // 256) * 256
        base = jnp.maximum(fb - 256, 0)
        for m in range(16):
            p0 = base + m * 16
            pos = p0 + i16
            keep = pos < fk
            sprd = (wid * 289 + pos * 37) & 8191
            ssrc[k][pl.ds(p0, 16)] = jnp.where(keep, ssrc[k][pl.ds(p0, 16)],
                                               sprd)
            sdst[k][pl.ds(p0, 16)] = jnp.where(keep, sdst[k][pl.ds(p0, 16)],
                                               sprd)
            sew[k][pl.ds(p0, 16)] = jnp.where(keep, sew[k][pl.ds(p0, 16)], 0.0)
        bb = pl.multiple_of((wid * NCHUNK + k) * CAP + ok, 256)
        pltpu.sync_copy(ssrc[k].at[pl.ds(0, FINAL)],
                        bsrc.at[pl.ds(bb, FINAL)])
        pltpu.sync_copy(sdst[k].at[pl.ds(0, FINAL)],
                        bdst.at[pl.ds(bb, FINAL)])
        pltpu.sync_copy(sew[k].at[pl.ds(0, FINAL)],
                        bew.at[pl.ds(bb, FINAL)])
        cv = jnp.where(i16 == k, ok + fb, cv)

    cbuf[...] = cv
    pltpu.sync_copy(cbuf, counts.at[wid])
    plsc.subcore_barrier()
    off = pl.multiple_of(s * 2560, 256)
    pltpu.sync_copy(degsh.at[pl.ds(off, 2560)],
                    degp.at[c, pl.ds(off, 2560)])


@functools.cache
def _bucket():
    return pl.kernel(
        _bucket_body,
        out_type=[
        jax.ShapeDtypeStruct((NT * NCHUNK * CAP,), jnp.int32),
        jax.ShapeDtypeStruct((NT * NCHUNK * CAP,), jnp.int32),
        jax.ShapeDtypeStruct((NT * NCHUNK * CAP,), jnp.float32),
        jax.ShapeDtypeStruct((NT, 16), jnp.int32),
        jax.ShapeDtypeStruct((2, N_PAD), jnp.float32),
    ],
        mesh=_mesh(),
        compiler_params=pltpu.CompilerParams(needs_layout_passes=False),
        scratch_types=[
        pltpu.VMEM((16, 128), jnp.int32),
        pltpu.VMEM((16, 128), jnp.int32),
        pltpu.VMEM((16, 128), jnp.float32),
        pltpu.VMEM((SCAP,), jnp.int32),
        pltpu.VMEM((SCAP,), jnp.int32),
        pltpu.VMEM((SCAP,), jnp.int32),
        pltpu.VMEM((SCAP,), jnp.int32),
        pltpu.VMEM((SCAP,), jnp.int32),
        pltpu.VMEM((SCAP,), jnp.int32),
        pltpu.VMEM((SCAP,), jnp.int32),
        pltpu.VMEM((SCAP,), jnp.int32),
        pltpu.VMEM((SCAP,), jnp.float32),
        pltpu.VMEM((SCAP,), jnp.float32),
        pltpu.VMEM((SCAP,), jnp.float32),
        pltpu.VMEM((SCAP,), jnp.float32),
        pltpu.VMEM((2560,), jnp.float32),
        pltpu.VMEM((16,), jnp.int32),
        pltpu.VMEM_SHARED((N_PAD,), jnp.float32),
        pltpu.SemaphoreType.DMA,
    ],
    )


def _spmm_body(hs, bsrc, bdst, bew, counts, agg,
               ssrc, sdst, sew, ra, rb, cbuf, acc, gA, gB, sA, sB):
    c = lax.axis_index("c")
    s = lax.axis_index("s")
    i16 = lax.iota(jnp.int32, 16)
    z16f = jnp.zeros((16,), jnp.float32)

    def _zero_ra(i, carry):
        for g in range(8):
            ra[i, pl.ds(g * 16, 16)] = z16f
        return carry

    def _scale(buf, lo):
        def srow(r, carry):
            for u in range(8):
                rr = r * 8 + u
                ewv = plsc.load_gather(
                    sew, [jnp.full((16,), 1, jnp.int32) * lo + rr])
                for g in range(8):
                    buf[rr, pl.ds(g * 16, 16)] = buf[rr, pl.ds(g * 16, 16)] * ewv
            return carry

        lax.fori_loop(0, GB // 8, srow, 0)

    def _stage(base, q):
        off = pl.multiple_of(base + q * 2048, 256)
        pltpu.sync_copy(bsrc.at[pl.ds(off, 2048)], ssrc)
        pltpu.sync_copy(bdst.at[pl.ds(off, 2048)], sdst)
        pltpu.sync_copy(bew.at[pl.ds(off, 2048)], sew)

    def _fire_gather(lo, buf, sem):
        for u in range(8):
            idx = ssrc[pl.ds(lo + u * 16, 16)]
            pltpu.async_copy(hs.at[idx], buf.at[pl.ds(u * 16, 16)], sem)

    def _wait_gather(buf, sem):
        for u in range(8):
            idx = ssrc[pl.ds(u * 16, 16)]
            pltpu.make_async_copy(hs.at[idx], buf.at[pl.ds(u * 16, 16)],
                                  sem).wait()

    def _fire_scatter(lo, buf, sem):
        for u in range(8):
            idx = sdst[pl.ds(lo + u * 16, 16)]
            pltpu.async_copy(buf.at[pl.ds(u * 16, 16)], acc.at[idx], sem,
                             add=True)

    def _wait_scatter(buf, sem):
        for u in range(8):
            idx = sdst[pl.ds(u * 16, 16)]
            pltpu.make_async_copy(buf.at[pl.ds(u * 16, 16)], acc.at[idx],
                                  sem).wait()

    for ci in range(2):
        k = c * 2 + ci
        plsc.subcore_barrier()
        lax.fori_loop(0, GB, _zero_ra, 0)
        for m in range(ACC_ROWS // 16 // 64):
            pltpu.sync_copy(ra.at[pl.ds(0, 64)], acc.at[pl.ds(
                pl.multiple_of(s * (ACC_ROWS // 16) + m * 64, 64), 64)])
        plsc.subcore_barrier()
        for bi in range(2):
            t = s * 2 + bi
            pltpu.sync_copy(counts.at[t], cbuf)
            cnt = cbuf[...].at[jnp.full((16,), k, jnp.int32)].get(
                mode="promise_in_bounds")[0]
            np2 = cnt // 256
            base = pl.multiple_of((t * NCHUNK + k) * CAP, 256)

            @pl.when(np2 > 0)
            def _process(k=k, np2=np2, base=base):
                _stage(base, 0)
                _fire_gather(0, ra, gA)
                _fire_gather(GB, rb, gB)

                def pair(p, carry):
                    lo = (p & 7) * 256
                    _wait_gather(ra, gA)
                    _scale(ra, lo)
                    _fire_scatter(lo, ra, sA)
                    _wait_gather(rb, gB)
                    _scale(rb, lo + GB)
                    _fire_scatter(lo + GB, rb, sB)

                    pn = p + 1

                    @pl.when(((pn & 7) == 0) & (pn < np2))
                    def _():
                        _stage(base, pn >> 3)

                    lon = (jnp.minimum(pn, np2 - 1) & 7) * 256
                    _wait_scatter(ra, sA)
                    _fire_gather(lon, ra, gA)
                    _wait_scatter(rb, sB)
                    _fire_gather(lon + GB, rb, gB)
                    return carry

                lax.fori_loop(0, np2, pair, 0)
                _wait_gather(ra, gA)
                _wait_gather(rb, gB)

        plsc.subcore_barrier()
        nr = ACC_ROWS // 16
        pltpu.sync_copy(acc.at[pl.ds(pl.multiple_of(s * nr, 64), nr)],
                        agg.at[pl.ds(pl.multiple_of(k * ACC_ROWS + s * nr, 64),
                                     nr)])


@functools.cache
def _spmm():
    return pl.kernel(
        _spmm_body,
        out_type=jax.ShapeDtypeStruct((NCHUNK * ACC_ROWS, IN_CH), jnp.float32),
        mesh=_mesh(),
        compiler_params=pltpu.CompilerParams(needs_layout_passes=False),
        scratch_types=[
        pltpu.VMEM((2048,), jnp.int32),
        pltpu.VMEM((2048,), jnp.int32),
        pltpu.VMEM((2048,), jnp.float32),
        pltpu.VMEM((GB, 128), jnp.float32),
        pltpu.VMEM((GB, 128), jnp.float32),
        pltpu.VMEM((16,), jnp.int32),
        pltpu.VMEM_SHARED((ACC_ROWS, 128), jnp.float32),
        pltpu.SemaphoreType.DMA,
        pltpu.SemaphoreType.DMA,
        pltpu.SemaphoreType.DMA,
        pltpu.SemaphoreType.DMA,
    ],
    )


def _mm1_body(x_ref, w_ref, degp_ref, hs_ref, dinv_ref):
    d = 1.0 + degp_ref[0, :] + degp_ref[1, :]
    dinv = lax.rsqrt(d)
    h = jnp.dot(x_ref[...], w_ref[...], preferred_element_type=jnp.float32)
    hs_ref[...] = h * dinv[:, None]
    dinv_ref[...] = dinv[:, None]


def _mm1(x, w, degp):
    BR = 512
    nb = pl.cdiv(N_TOT, BR)
    return pl.pallas_call(
        _mm1_body,
        grid=(nb,),
        in_specs=[
            pl.BlockSpec((BR, IN_CH), lambda i: (i, 0)),
            pl.BlockSpec((IN_CH, HID), lambda i: (0, 0)),
            pl.BlockSpec((2, BR), lambda i: (0, i)),
        ],
        out_specs=[
            pl.BlockSpec((BR, HID), lambda i: (i, 0)),
            pl.BlockSpec((BR, 1), lambda i: (i, 0)),
        ],
        out_shape=[
            jax.ShapeDtypeStruct((N_TOT, HID), jnp.float32),
            jax.ShapeDtypeStruct((N_TOT, 1), jnp.float32),
        ],
    )(x, w, degp)


def _mm2_body(agg_ref, hs_ref, dinv_ref, bc_ref, g_ref, be_ref, w_ref,
              h1_ref, hs2_ref):
    dinv = dinv_ref[...]
    pre = dinv * (agg_ref[...] + hs_ref[...]) + bc_ref[...]
    h1 = jnp.maximum(pre, 0.0)
    h1 = g_ref[...] * (h1 * _BN_SCALE) + be_ref[...]
    h1_ref[...] = h1
    hs2_ref[...] = jnp.dot(
        h1, w_ref[...], preferred_element_type=jnp.float32) * dinv


def _mm2(agg, hs, dinv, bc, g, be, w):
    BR = 512
    nb = pl.cdiv(N_TOT, BR)
    return pl.pallas_call(
        _mm2_body,
        grid=(nb,),
        in_specs=[
            pl.BlockSpec((BR, HID), lambda i: (i, 0)),
            pl.BlockSpec((BR, HID), lambda i: (i, 0)),
            pl.BlockSpec((BR, 1), lambda i: (i, 0)),
            pl.BlockSpec((1, HID), lambda i: (0, 0)),
            pl.BlockSpec((1, HID), lambda i: (0, 0)),
            pl.BlockSpec((1, HID), lambda i: (0, 0)),
            pl.BlockSpec((HID, HID), lambda i: (0, 0)),
        ],
        out_specs=[
            pl.BlockSpec((BR, HID), lambda i: (i, 0)),
            pl.BlockSpec((BR, HID), lambda i: (i, 0)),
        ],
        out_shape=[
            jax.ShapeDtypeStruct((N_TOT, HID), jnp.float32),
            jax.ShapeDtypeStruct((N_TOT, HID), jnp.float32),
        ],
    )(agg, hs, dinv, bc, g, be, w)


def _sigm(v):
    return jax.nn.sigmoid(v)


def _lstm_head_body(h1_ref, agg2_ref, hs2_ref, dinv_ref, bc2_ref, g2_ref,
                    be2_ref, wih1_ref, whh1_ref, b1_ref, wih2_ref, whh2_ref,
                    b2_ref, x_ref, wl1_ref, bl1_ref, wl2_ref, bl2_ref, o_ref):
    BR = h1_ref.shape[1]
    h1 = h1_ref[...]
    dinv = dinv_ref[...]
    h2 = dinv * (agg2_ref[...] + hs2_ref[...]) + bc2_ref[...]
    h2 = jnp.maximum(h2, 0.0)
    h2 = g2_ref[...] * (h2 * _BN_SCALE) + be2_ref[...]
    wih1 = wih1_ref[...]
    whh1 = whh1_ref[...]
    b1 = b1_ref[...]
    wih2 = wih2_ref[...]
    whh2 = whh2_ref[...]
    b2 = b2_ref[...]
    hA = jnp.zeros((BR, HID), jnp.float32)
    cA = jnp.zeros((BR, HID), jnp.float32)
    hB = jnp.zeros((BR, HID), jnp.float32)
    cB = jnp.zeros((BR, HID), jnp.float32)
    for t in range(WINDOW):
        xt = jnp.concatenate([h1[t], h2[t]], axis=1)
        gates = (jnp.dot(xt, wih1.T, preferred_element_type=jnp.float32)
                 + jnp.dot(hA, whh1.T, preferred_element_type=jnp.float32)
                 + b1)
        ig, fg, gg, og = jnp.split(gates, 4, axis=1)
        cA = _sigm(fg) * cA + _sigm(ig) * jnp.tanh(gg)
        hA = _sigm(og) * jnp.tanh(cA)
        gates = (jnp.dot(hA, wih2.T, preferred_element_type=jnp.float32)
                 + jnp.dot(hB, whh2.T, preferred_element_type=jnp.float32)
                 + b2)
        ig, fg, gg, og = jnp.split(gates, 4, axis=1)
        cB = _sigm(fg) * cB + _sigm(ig) * jnp.tanh(gg)
        hB = _sigm(og) * jnp.tanh(cB)
    x4 = x_ref[...]
    S = jnp.concatenate(
        [x4[0], x4[1][:, IN_CH - 1:], x4[2][:, IN_CH - 1:],
         x4[3][:, IN_CH - 1:]], axis=1)
    hcat = jnp.maximum(jnp.concatenate([hA, hB, S], axis=1), 0.0)
    z = jnp.maximum(
        jnp.dot(hcat, wl1_ref[...].T, preferred_element_type=jnp.float32)
        + bl1_ref[...], 0.0)
    o_ref[...] = (jnp.dot(z, wl2_ref[...].T,
                          preferred_element_type=jnp.float32) + bl2_ref[...])


def _lstm_head(h1r, agg2r, hs2r, dinvr, bc2, g2, be2, Wih1, Whh1, b1,
               Wih2, Whh2, b2, xr, Wl1, bl1, Wl2, bl2):
    BR = 512
    nb = pl.cdiv(NUM_NODES, BR)
    d1 = 2 * HID + IN_CH + WINDOW - 1
    full = lambda shape: pl.BlockSpec(shape, lambda i: tuple(0 for _ in shape))
    return pl.pallas_call(
        _lstm_head_body,
        grid=(nb,),
        in_specs=[
            pl.BlockSpec((WINDOW, BR, HID), lambda i: (0, i, 0)),
            pl.BlockSpec((WINDOW, BR, HID), lambda i: (0, i, 0)),
            pl.BlockSpec((WINDOW, BR, HID), lambda i: (0, i, 0)),
            pl.BlockSpec((WINDOW, BR, 1), lambda i: (0, i, 0)),
            full((1, HID)),
            full((1, HID)),
            full((1, HID)),
            full((4 * HID, 2 * HID)),
            full((4 * HID, HID)),
            full((1, 4 * HID)),
            full((4 * HID, HID)),
            full((4 * HID, HID)),
            full((1, 4 * HID)),
            pl.BlockSpec((WINDOW, BR, IN_CH), lambda i: (0, i, 0)),
            full((HID, d1)),
            full((1, HID)),
            full((D_D, HID)),
            full((1, D_D)),
        ],
        out_specs=pl.BlockSpec((BR, D_D), lambda i: (i, 0)),
        out_shape=jax.ShapeDtypeStruct((NUM_NODES, D_D), jnp.float32),
    )(h1r, agg2r, hs2r, dinvr, bc2, g2, be2, Wih1, Whh1, b1, Wih2, Whh2, b2,
      xr, Wl1, bl1, Wl2, bl2)


def kernel(x, edge_index, edge_weight, Wc1, bc1, Wc2, bc2, g1, be1, g2, be2,
           Wih1, Whh1, bih1, bhh1, Wih2, Whh2, bih2, bhh2, Wl1, bl1, Wl2, bl2):
    e = edge_weight.shape[0]
    pad = EPAD - e
    spread = jnp.arange(pad, dtype=edge_index.dtype) % N_TOT
    row2 = jnp.concatenate([edge_index[0], spread]).reshape(-1, 128)
    col2 = jnp.concatenate([edge_index[1], spread]).reshape(-1, 128)
    ew2 = jnp.concatenate(
        [edge_weight, jnp.zeros((pad,), edge_weight.dtype)]).reshape(-1, 128)

    bsrc, bdst, bew, counts, degp = _bucket()(row2, col2, ew2)

    unpad = lambda a: a.reshape(NCHUNK, ACC_ROWS, IN_CH)[:, :NUM_NODES, :]
    hs1, dinv = _mm1(x, Wc1, degp)
    agg1 = unpad(_spmm()(hs1, bsrc, bdst, bew, counts)).reshape(N_TOT, IN_CH)
    h1, hs2 = _mm2(agg1, hs1, dinv, bc1.reshape(1, -1), g1.reshape(1, -1),
                   be1.reshape(1, -1), Wc2)
    agg2 = unpad(_spmm()(hs2, bsrc, bdst, bew, counts)).reshape(N_TOT, IN_CH)

    r4 = lambda a: a.reshape(WINDOW, NUM_NODES, -1)
    out = _lstm_head(
        r4(h1), r4(agg2), r4(hs2), r4(dinv), bc2.reshape(1, -1),
        g2.reshape(1, -1), be2.reshape(1, -1), Wih1, Whh1,
        (bih1 + bhh1).reshape(1, -1), Wih2, Whh2,
        (bih2 + bhh2).reshape(1, -1), r4(x), Wl1, bl1.reshape(1, -1),
        Wl2, bl2.reshape(1, -1))
    return out


# speedup vs baseline: 13.6992x; 1.2073x over previous
import functools
import math

import jax
import jax.numpy as jnp
from jax import lax
from jax.experimental import pallas as pl
from jax.experimental.pallas import tpu as pltpu
from jax.experimental.pallas import tpu_sc as plsc

IN_CH = 128
HID = 128
NUM_NODES = 10000
WINDOW = 4
D_D = 7
N_TOT = NUM_NODES * WINDOW
N_PAD = 40960
EPAD = 524288
NT = 32
NCHUNK = 4
EPT = EPAD // NT
EB = 2048
NBLOCKS = EPT // EB
SCAP = 2320
FLUSH = 2048
FINAL = 2304
CAP = 18688
CAPR = CAP // 128
GB = 128
ACC_ROWS = 10240
_BN_SCALE = 1.0 / math.sqrt(1.0 + 1e-5)

@functools.cache
def _mesh():
    return plsc.VectorSubcoreMesh(core_axis_name="c", subcore_axis_name="s")


def _bucket_body(row2, col2, ew2, bsrc, bdst, bew, counts, degp,
                 rstg, cstg, wstg,
                 ssrc0, ssrc1, ssrc2, ssrc3,
                 sdst0, sdst1, sdst2, sdst3,
                 sew0, sew1, sew2, sew3,
                 zbuf, cbuf, degsh, dsem):
    ssrc = [ssrc0, ssrc1, ssrc2, ssrc3]
    sdst = [sdst0, sdst1, sdst2, sdst3]
    sew = [sew0, sew1, sew2, sew3]
    c = lax.axis_index("c")
    s = lax.axis_index("s")
    wid = c * 16 + s
    i16 = lax.iota(jnp.int32, 16)
    z16f = jnp.zeros((16,), jnp.float32)

    def _zb(i, carry):
        zbuf[pl.ds(i * 16, 16)] = z16f
        return carry

    lax.fori_loop(0, 2560 // 16, _zb, 0)
    pltpu.sync_copy(zbuf, degsh.at[pl.ds(pl.multiple_of(s * 2560, 256), 2560)])
    plsc.subcore_barrier()

    def process_block(blk, carry):
        r0 = wid * (EPT // 128) + blk * 16
        pltpu.sync_copy(row2.at[pl.ds(r0, 16)], rstg)
        pltpu.sync_copy(col2.at[pl.ds(r0, 16)], cstg)
        pltpu.sync_copy(ew2.at[pl.ds(r0, 16)], wstg)
        for j in range(16):
            pltpu.async_copy(wstg.at[j], degsh.at[cstg.at[j]], dsem, add=True)

        def batch(b, fc):
            j = b // 8
            o = (b % 8) * 16
            rvec = rstg[j, pl.ds(o, 16)]
            cvec = cstg[j, pl.ds(o, 16)]
            wvec = wstg[j, pl.ds(o, 16)]
            fc = list(fc)
            for k in range(NCHUNK):
                fk = fc[k]
                ok = fc[NCHUNK + k]
                m = (cvec >= k * NUM_NODES) & (cvec < (k + 1) * NUM_NODES)
                mi = m.astype(jnp.int32)
                csum = mi
                for st in (1, 2, 4, 8):
                    g = csum.at[jnp.maximum(i16 - st, 0)].get(
                        mode="promise_in_bounds")
                    csum = csum + jnp.where(i16 >= st, g, 0)
                pos = fk + csum - mi
                plsc.store_scatter(ssrc[k], [pos], rvec, mask=m)
                plsc.store_scatter(sdst[k], [pos], cvec - k * NUM_NODES,
                                   mask=m)
                plsc.store_scatter(sew[k], [pos], wvec, mask=m)
                fk = fk + csum[15]
                do = fk >= FLUSH

                @pl.when(do)
                def _flush(k=k, ok=ok):
                    bb = pl.multiple_of((wid * NCHUNK + k) * CAP + ok, 256)
                    pltpu.sync_copy(ssrc[k].at[pl.ds(0, FLUSH)],
                                    bsrc.at[pl.ds(bb, FLUSH)])
                    pltpu.sync_copy(sdst[k].at[pl.ds(0, FLUSH)],
                                    bdst.at[pl.ds(bb, FLUSH)])
                    pltpu.sync_copy(sew[k].at[pl.ds(0, FLUSH)],
                                    bew.at[pl.ds(bb, FLUSH)])
                    vs = ssrc[k][pl.ds(FLUSH, 16)]
                    vd = sdst[k][pl.ds(FLUSH, 16)]
                    vw = sew[k][pl.ds(FLUSH, 16)]
                    ssrc[k][pl.ds(0, 16)] = vs
                    sdst[k][pl.ds(0, 16)] = vd
                    sew[k][pl.ds(0, 16)] = vw

                fc[k] = jnp.where(do, fk - FLUSH, fk)
                fc[NCHUNK + k] = jnp.where(do, ok + FLUSH, ok)
            return tuple(fc)

        carry = lax.fori_loop(0, EB // 16, batch, carry)
        for j in range(16):
            pltpu.make_async_copy(wstg.at[j], degsh.at[cstg.at[j]], dsem).wait()
        return carry

    carry = (0, 0, 0, 0, 0, 0, 0, 0)
    for _blk in range(NBLOCKS):
        carry = process_block(_blk, carry)

    cv = jnp.zeros((16,), jnp.int32)
    for k in range(NCHUNK):
        fk = carry[k]
        ok = carry[NCHUNK + k]
        fb = ((fk + 255) // 256) * 256
        base = jnp.maximum(fb - 256, 0)
        for m in range(16):
            p0 = base + m * 16
            pos = p0 + i16
            keep = pos < fk
            sprd = (wid * 289 + pos * 37) & 8191
            ssrc[k][pl.ds(p0, 16)] = jnp.where(keep, ssrc[k][pl.ds(p0, 16)],
                                               sprd)
            sdst[k][pl.ds(p0, 16)] = jnp.where(keep, sdst[k][pl.ds(p0, 16)],
                                               sprd)
            sew[k][pl.ds(p0, 16)] = jnp.where(keep, sew[k][pl.ds(p0, 16)], 0.0)
        bb = pl.multiple_of((wid * NCHUNK + k) * CAP + ok, 256)
        pltpu.sync_copy(ssrc[k].at[pl.ds(0, FINAL)],
                        bsrc.at[pl.ds(bb, FINAL)])
        pltpu.sync_copy(sdst[k].at[pl.ds(0, FINAL)],
                        bdst.at[pl.ds(bb, FINAL)])
        pltpu.sync_copy(sew[k].at[pl.ds(0, FINAL)],
                        bew.at[pl.ds(bb, FINAL)])
        cv = jnp.where(i16 == k, ok + fb, cv)

    cbuf[...] = cv
    pltpu.sync_copy(cbuf, counts.at[wid])
    plsc.subcore_barrier()
    off = pl.multiple_of(s * 2560, 256)
    pltpu.sync_copy(degsh.at[pl.ds(off, 2560)],
                    degp.at[c, pl.ds(off, 2560)])


@functools.cache
def _bucket():
    return pl.kernel(
        _bucket_body,
        out_type=[
        jax.ShapeDtypeStruct((NT * NCHUNK * CAP,), jnp.int32),
        jax.ShapeDtypeStruct((NT * NCHUNK * CAP,), jnp.int32),
        jax.ShapeDtypeStruct((NT * NCHUNK * CAP,), jnp.float32),
        jax.ShapeDtypeStruct((NT, 16), jnp.int32),
        jax.ShapeDtypeStruct((2, N_PAD), jnp.float32),
    ],
        mesh=_mesh(),
        compiler_params=pltpu.CompilerParams(needs_layout_passes=False),
        scratch_types=[
        pltpu.VMEM((16, 128), jnp.int32),
        pltpu.VMEM((16, 128), jnp.int32),
        pltpu.VMEM((16, 128), jnp.float32),
        pltpu.VMEM((SCAP,), jnp.int32),
        pltpu.VMEM((SCAP,), jnp.int32),
        pltpu.VMEM((SCAP,), jnp.int32),
        pltpu.VMEM((SCAP,), jnp.int32),
        pltpu.VMEM((SCAP,), jnp.int32),
        pltpu.VMEM((SCAP,), jnp.int32),
        pltpu.VMEM((SCAP,), jnp.int32),
        pltpu.VMEM((SCAP,), jnp.int32),
        pltpu.VMEM((SCAP,), jnp.float32),
        pltpu.VMEM((SCAP,), jnp.float32),
        pltpu.VMEM((SCAP,), jnp.float32),
        pltpu.VMEM((SCAP,), jnp.float32),
        pltpu.VMEM((2560,), jnp.float32),
        pltpu.VMEM((16,), jnp.int32),
        pltpu.VMEM_SHARED((N_PAD,), jnp.float32),
        pltpu.SemaphoreType.DMA,
    ],
    )


def _spmm_body(hs, bsrc, bdst, bew, counts, agg,
               ssrc, sdst, sew, ra, rb, cbuf, acc, gA, gB, sA, sB):
    c = lax.axis_index("c")
    s = lax.axis_index("s")
    i16 = lax.iota(jnp.int32, 16)
    z16f = jnp.zeros((16,), jnp.float32)

    def _zero_ra(i, carry):
        for g in range(8):
            ra[i, pl.ds(g * 16, 16)] = z16f
        return carry

    def _scale(buf, lo):
        def srow(r, carry):
            for u in range(8):
                rr = r * 8 + u
                ewv = plsc.load_gather(
                    sew, [jnp.full((16,), 1, jnp.int32) * lo + rr])
                for g in range(8):
                    buf[rr, pl.ds(g * 16, 16)] = buf[rr, pl.ds(g * 16, 16)] * ewv
            return carry

        lax.fori_loop(0, GB // 8, srow, 0)

    def _stage(base, q):
        off = pl.multiple_of(base + q * 2048, 256)
        pltpu.sync_copy(bsrc.at[pl.ds(off, 2048)], ssrc)
        pltpu.sync_copy(bdst.at[pl.ds(off, 2048)], sdst)
        pltpu.sync_copy(bew.at[pl.ds(off, 2048)], sew)

    def _fire_gather(lo, buf, sem):
        for u in range(8):
            idx = ssrc[pl.ds(lo + u * 16, 16)]
            pltpu.async_copy(hs.at[idx], buf.at[pl.ds(u * 16, 16)], sem)

    def _wait_gather(buf, sem):
        for u in range(8):
            idx = ssrc[pl.ds(u * 16, 16)]
            pltpu.make_async_copy(hs.at[idx], buf.at[pl.ds(u * 16, 16)],
                                  sem).wait()

    def _fire_scatter(lo, buf, sem):
        for u in range(8):
            idx = sdst[pl.ds(lo + u * 16, 16)]
            pltpu.async_copy(buf.at[pl.ds(u * 16, 16)], acc.at[idx], sem,
                             add=True)

    def _wait_scatter(buf, sem):
        for u in range(8):
            idx = sdst[pl.ds(u * 16, 16)]
            pltpu.make_async_copy(buf.at[pl.ds(u * 16, 16)], acc.at[idx],
                                  sem).wait()

    for ci in range(2):
        k = c * 2 + ci
        plsc.subcore_barrier()
        lax.fori_loop(0, GB, _zero_ra, 0)
        for m in range(ACC_ROWS // 16 // 64):
            pltpu.sync_copy(ra.at[pl.ds(0, 64)], acc.at[pl.ds(
                pl.multiple_of(s * (ACC_ROWS // 16) + m * 64, 64), 64)])
        plsc.subcore_barrier()
        for bi in range(2):
            t = s * 2 + bi
            pltpu.sync_copy(counts.at[t], cbuf)
            cnt = cbuf[...].at[jnp.full((16,), k, jnp.int32)].get(
                mode="promise_in_bounds")[0]
            np2 = cnt // 256
            base = pl.multiple_of((t * NCHUNK + k) * CAP, 256)

            @pl.when(np2 > 0)
            def _process(k=k, np2=np2, base=base):
                _stage(base, 0)
                _fire_gather(0, ra, gA)
                _fire_gather(GB, rb, gB)

                def pair(p, carry):
                    lo = (p & 7) * 256
                    _wait_gather(ra, gA)
                    _scale(ra, lo)
                    _fire_scatter(lo, ra, sA)
                    _wait_gather(rb, gB)
                    _scale(rb, lo + GB)
                    _fire_scatter(lo + GB, rb, sB)

                    pn = p + 1

                    @pl.when(((pn & 7) == 0) & (pn < np2))
                    def _():
                        _stage(base, pn >> 3)

                    lon = (jnp.minimum(pn, np2 - 1) & 7) * 256
                    _wait_scatter(ra, sA)
                    _fire_gather(lon, ra, gA)
                    _wait_scatter(rb, sB)
                    _fire_gather(lon + GB, rb, gB)
                    return carry

                lax.fori_loop(0, np2, pair, 0)
                _wait_gather(ra, gA)
                _wait_gather(rb, gB)

        plsc.subcore_barrier()
        nr = ACC_ROWS // 16
        pltpu.sync_copy(acc.at[pl.ds(pl.multiple_of(s * nr, 64), nr)],
                        agg.at[pl.ds(pl.multiple_of(k * ACC_ROWS + s * nr, 64),
                                     nr)])


@functools.cache
def _spmm():
    return pl.kernel(
        _spmm_body,
        out_type=jax.ShapeDtypeStruct((NCHUNK * ACC_ROWS, IN_CH), jnp.float32),
        mesh=_mesh(),
        compiler_params=pltpu.CompilerParams(needs_layout_passes=False),
        scratch_types=[
        pltpu.VMEM((2048,), jnp.int32),
        pltpu.VMEM((2048,), jnp.int32),
        pltpu.VMEM((2048,), jnp.float32),
        pltpu.VMEM((GB, 128), jnp.float32),
        pltpu.VMEM((GB, 128), jnp.float32),
        pltpu.VMEM((16,), jnp.int32),
        pltpu.VMEM_SHARED((ACC_ROWS, 128), jnp.float32),
        pltpu.SemaphoreType.DMA,
        pltpu.SemaphoreType.DMA,
        pltpu.SemaphoreType.DMA,
        pltpu.SemaphoreType.DMA,
    ],
    )


def _mm1_body(x_ref, w_ref, degp_ref, hs_ref, dinv_ref):
    d = 1.0 + degp_ref[0, :] + degp_ref[1, :]
    dinv = lax.rsqrt(d)
    h = jnp.dot(x_ref[...], w_ref[...], preferred_element_type=jnp.float32)
    hs_ref[...] = h * dinv[:, None]
    dinv_ref[...] = dinv[:, None]


def _mm1(x, w, degp):
    BR = 512
    nb = pl.cdiv(N_TOT, BR)
    return pl.pallas_call(
        _mm1_body,
        grid=(nb,),
        in_specs=[
            pl.BlockSpec((BR, IN_CH), lambda i: (i, 0)),
            pl.BlockSpec((IN_CH, HID), lambda i: (0, 0)),
            pl.BlockSpec((2, BR), lambda i: (0, i)),
        ],
        out_specs=[
            pl.BlockSpec((BR, HID), lambda i: (i, 0)),
            pl.BlockSpec((BR, 1), lambda i: (i, 0)),
        ],
        out_shape=[
            jax.ShapeDtypeStruct((N_TOT, HID), jnp.float32),
            jax.ShapeDtypeStruct((N_TOT, 1), jnp.float32),
        ],
    )(x, w, degp)


def _mm2_body(agg_ref, hs_ref, dinv_ref, bc_ref, g_ref, be_ref, w_ref,
              h1_ref, hs2_ref):
    dinv = dinv_ref[...]
    pre = dinv * (agg_ref[...] + hs_ref[...]) + bc_ref[...]
    h1 = jnp.maximum(pre, 0.0)
    h1 = g_ref[...] * (h1 * _BN_SCALE) + be_ref[...]
    h1_ref[...] = h1
    hs2_ref[...] = jnp.dot(
        h1, w_ref[...], preferred_element_type=jnp.float32) * dinv


def _mm2(agg, hs, dinv, bc, g, be, w):
    BR = 512
    nb = pl.cdiv(N_TOT, BR)
    return pl.pallas_call(
        _mm2_body,
        grid=(nb,),
        in_specs=[
            pl.BlockSpec((BR, HID), lambda i: (i, 0)),
            pl.BlockSpec((BR, HID), lambda i: (i, 0)),
            pl.BlockSpec((BR, 1), lambda i: (i, 0)),
            pl.BlockSpec((1, HID), lambda i: (0, 0)),
            pl.BlockSpec((1, HID), lambda i: (0, 0)),
            pl.BlockSpec((1, HID), lambda i: (0, 0)),
            pl.BlockSpec((HID, HID), lambda i: (0, 0)),
        ],
        out_specs=[
            pl.BlockSpec((BR, HID), lambda i: (i, 0)),
            pl.BlockSpec((BR, HID), lambda i: (i, 0)),
        ],
        out_shape=[
            jax.ShapeDtypeStruct((N_TOT, HID), jnp.float32),
            jax.ShapeDtypeStruct((N_TOT, HID), jnp.float32),
        ],
    )(agg, hs, dinv, bc, g, be, w)


def _sigm(v):
    return jax.nn.sigmoid(v)


def _lstm_head_body(h1_ref, agg2_ref, hs2_ref, dinv_ref, bc2_ref, g2_ref,
                    be2_ref, wih1_ref, whh1_ref, b1_ref, wih2_ref, whh2_ref,
                    b2_ref, x_ref, wl1_ref, bl1_ref, wl2_ref, bl2_ref, o_ref):
    BR = h1_ref.shape[1]
    h1 = h1_ref[...]
    dinv = dinv_ref[...]
    h2 = dinv * (agg2_ref[...] + hs2_ref[...]) + bc2_ref[...]
    h2 = jnp.maximum(h2, 0.0)
    h2 = g2_ref[...] * (h2 * _BN_SCALE) + be2_ref[...]
    bf = jnp.bfloat16
    wih1 = wih1_ref[...].astype(bf)
    whh1 = whh1_ref[...].astype(bf)
    b1 = b1_ref[...]
    wih2 = wih2_ref[...].astype(bf)
    whh2 = whh2_ref[...].astype(bf)
    b2 = b2_ref[...]
    hA = jnp.zeros((BR, HID), jnp.float32)
    cA = jnp.zeros((BR, HID), jnp.float32)
    hB = jnp.zeros((BR, HID), jnp.float32)
    cB = jnp.zeros((BR, HID), jnp.float32)
    for t in range(WINDOW):
        xt = jnp.concatenate([h1[t], h2[t]], axis=1).astype(bf)
        gates = (jnp.dot(xt, wih1.T, preferred_element_type=jnp.float32)
                 + jnp.dot(hA.astype(bf), whh1.T,
                           preferred_element_type=jnp.float32)
                 + b1)
        ig, fg, gg, og = jnp.split(gates, 4, axis=1)
        cA = _sigm(fg) * cA + _sigm(ig) * jnp.tanh(gg)
        hA = _sigm(og) * jnp.tanh(cA)
        gates = (jnp.dot(hA.astype(bf), wih2.T,
                         preferred_element_type=jnp.float32)
                 + jnp.dot(hB.astype(bf), whh2.T,
                           preferred_element_type=jnp.float32)
                 + b2)
        ig, fg, gg, og = jnp.split(gates, 4, axis=1)
        cB = _sigm(fg) * cB + _sigm(ig) * jnp.tanh(gg)
        hB = _sigm(og) * jnp.tanh(cB)
    x4 = x_ref[...]
    S = jnp.concatenate(
        [x4[0], x4[1][:, IN_CH - 1:], x4[2][:, IN_CH - 1:],
         x4[3][:, IN_CH - 1:]], axis=1)
    hcat = jnp.maximum(jnp.concatenate([hA, hB, S], axis=1), 0.0)
    z = jnp.maximum(
        jnp.dot(hcat, wl1_ref[...].T, preferred_element_type=jnp.float32)
        + bl1_ref[...], 0.0)
    o_ref[...] = (jnp.dot(z, wl2_ref[...].T,
                          preferred_element_type=jnp.float32) + bl2_ref[...])


def _lstm_head(h1r, agg2r, hs2r, dinvr, bc2, g2, be2, Wih1, Whh1, b1,
               Wih2, Whh2, b2, xr, Wl1, bl1, Wl2, bl2):
    BR = 512
    nb = pl.cdiv(NUM_NODES, BR)
    d1 = 2 * HID + IN_CH + WINDOW - 1
    full = lambda shape: pl.BlockSpec(shape, lambda i: tuple(0 for _ in shape))
    return pl.pallas_call(
        _lstm_head_body,
        grid=(nb,),
        in_specs=[
            pl.BlockSpec((WINDOW, BR, HID), lambda i: (0, i, 0)),
            pl.BlockSpec((WINDOW, BR, HID), lambda i: (0, i, 0)),
            pl.BlockSpec((WINDOW, BR, HID), lambda i: (0, i, 0)),
            pl.BlockSpec((WINDOW, BR, 1), lambda i: (0, i, 0)),
            full((1, HID)),
            full((1, HID)),
            full((1, HID)),
            full((4 * HID, 2 * HID)),
            full((4 * HID, HID)),
            full((1, 4 * HID)),
            full((4 * HID, HID)),
            full((4 * HID, HID)),
            full((1, 4 * HID)),
            pl.BlockSpec((WINDOW, BR, IN_CH), lambda i: (0, i, 0)),
            full((HID, d1)),
            full((1, HID)),
            full((D_D, HID)),
            full((1, D_D)),
        ],
        out_specs=pl.BlockSpec((BR, D_D), lambda i: (i, 0)),
        out_shape=jax.ShapeDtypeStruct((NUM_NODES, D_D), jnp.float32),
    )(h1r, agg2r, hs2r, dinvr, bc2, g2, be2, Wih1, Whh1, b1, Wih2, Whh2, b2,
      xr, Wl1, bl1, Wl2, bl2)


def kernel(x, edge_index, edge_weight, Wc1, bc1, Wc2, bc2, g1, be1, g2, be2,
           Wih1, Whh1, bih1, bhh1, Wih2, Whh2, bih2, bhh2, Wl1, bl1, Wl2, bl2):
    e = edge_weight.shape[0]
    pad = EPAD - e
    spread = (jnp.arange(pad, dtype=edge_index.dtype) * N_TOT) // pad
    row2 = jnp.concatenate([edge_index[0], spread]).reshape(-1, 128)
    col2 = jnp.concatenate([edge_index[1], spread]).reshape(-1, 128)
    ew2 = jnp.concatenate(
        [edge_weight, jnp.zeros((pad,), edge_weight.dtype)]).reshape(-1, 128)

    bsrc, bdst, bew, counts, degp = _bucket()(row2, col2, ew2)

    unpad = lambda a: a.reshape(NCHUNK, ACC_ROWS, IN_CH)[:, :NUM_NODES, :]
    hs1, dinv = _mm1(x, Wc1, degp)
    agg1 = unpad(_spmm()(hs1, bsrc, bdst, bew, counts)).reshape(N_TOT, IN_CH)
    h1, hs2 = _mm2(agg1, hs1, dinv, bc1.reshape(1, -1), g1.reshape(1, -1),
                   be1.reshape(1, -1), Wc2)
    agg2 = unpad(_spmm()(hs2, bsrc, bdst, bew, counts)).reshape(N_TOT, IN_CH)

    r4 = lambda a: a.reshape(WINDOW, NUM_NODES, -1)
    out = _lstm_head(
        r4(h1), r4(agg2), r4(hs2), r4(dinv), bc2.reshape(1, -1),
        g2.reshape(1, -1), be2.reshape(1, -1), Wih1, Whh1,
        (bih1 + bhh1).reshape(1, -1), Wih2, Whh2,
        (bih2 + bhh2).reshape(1, -1), r4(x), Wl1, bl1.reshape(1, -1),
        Wl2, bl2.reshape(1, -1))
    return out


# speedup vs baseline: 13.7361x; 1.0027x over previous
import functools
import math

import jax
import jax.numpy as jnp
from jax import lax
from jax.experimental import pallas as pl
from jax.experimental.pallas import tpu as pltpu
from jax.experimental.pallas import tpu_sc as plsc

IN_CH = 128
HID = 128
NUM_NODES = 10000
WINDOW = 4
D_D = 7
N_TOT = NUM_NODES * WINDOW
N_PAD = 40960
EPAD = 524288
NT = 32
NCHUNK = 4
EPT = EPAD // NT
EB = 2048
NBLOCKS = EPT // EB
SCAP = 2320
FLUSH = 2048
FINAL = 2304
CAP = 18688
CAPR = CAP // 128
GB = 128
ACC_ROWS = 10240
_BN_SCALE = 1.0 / math.sqrt(1.0 + 1e-5)

@functools.cache
def _mesh():
    return plsc.VectorSubcoreMesh(core_axis_name="c", subcore_axis_name="s")


def _bucket_body(row2, col2, ew2, bsrc, bdst, bew, counts, degp,
                 rstg, cstg, wstg,
                 ssrc0, ssrc1, ssrc2, ssrc3,
                 sdst0, sdst1, sdst2, sdst3,
                 sew0, sew1, sew2, sew3,
                 zbuf, cbuf, degsh, dsem):
    ssrc = [ssrc0, ssrc1, ssrc2, ssrc3]
    sdst = [sdst0, sdst1, sdst2, sdst3]
    sew = [sew0, sew1, sew2, sew3]
    c = lax.axis_index("c")
    s = lax.axis_index("s")
    wid = c * 16 + s
    i16 = lax.iota(jnp.int32, 16)
    z16f = jnp.zeros((16,), jnp.float32)

    def _zb(i, carry):
        zbuf[pl.ds(i * 16, 16)] = z16f
        return carry

    lax.fori_loop(0, 2560 // 16, _zb, 0)
    pltpu.sync_copy(zbuf, degsh.at[pl.ds(pl.multiple_of(s * 2560, 256), 2560)])
    plsc.subcore_barrier()

    def process_block(blk, carry):
        r0 = wid * (EPT // 128) + blk * 16
        pltpu.sync_copy(row2.at[pl.ds(r0, 16)], rstg)
        pltpu.sync_copy(col2.at[pl.ds(r0, 16)], cstg)
        pltpu.sync_copy(ew2.at[pl.ds(r0, 16)], wstg)
        for j in range(16):
            pltpu.async_copy(wstg.at[j], degsh.at[cstg.at[j]], dsem, add=True)

        def batch(b, fc):
            j = b // 8
            o = (b % 8) * 16
            rvec = rstg[j, pl.ds(o, 16)]
            cvec = cstg[j, pl.ds(o, 16)]
            wvec = wstg[j, pl.ds(o, 16)]
            fc = list(fc)
            for k in range(NCHUNK):
                fk = fc[k]
                ok = fc[NCHUNK + k]
                m = (cvec >= k * NUM_NODES) & (cvec < (k + 1) * NUM_NODES)
                mi = m.astype(jnp.int32)
                csum = mi
                for st in (1, 2, 4, 8):
                    g = csum.at[jnp.maximum(i16 - st, 0)].get(
                        mode="promise_in_bounds")
                    csum = csum + jnp.where(i16 >= st, g, 0)
                pos = fk + csum - mi
                plsc.store_scatter(ssrc[k], [pos], rvec, mask=m)
                plsc.store_scatter(sdst[k], [pos], cvec - k * NUM_NODES,
                                   mask=m)
                plsc.store_scatter(sew[k], [pos], wvec, mask=m)
                fk = fk + csum[15]
                do = fk >= FLUSH

                @pl.when(do)
                def _flush(k=k, ok=ok):
                    bb = pl.multiple_of((wid * NCHUNK + k) * CAP + ok, 256)
                    pltpu.sync_copy(ssrc[k].at[pl.ds(0, FLUSH)],
                                    bsrc.at[pl.ds(bb, FLUSH)])
                    pltpu.sync_copy(sdst[k].at[pl.ds(0, FLUSH)],
                                    bdst.at[pl.ds(bb, FLUSH)])
                    pltpu.sync_copy(sew[k].at[pl.ds(0, FLUSH)],
                                    bew.at[pl.ds(bb, FLUSH)])
                    vs = ssrc[k][pl.ds(FLUSH, 16)]
                    vd = sdst[k][pl.ds(FLUSH, 16)]
                    vw = sew[k][pl.ds(FLUSH, 16)]
                    ssrc[k][pl.ds(0, 16)] = vs
                    sdst[k][pl.ds(0, 16)] = vd
                    sew[k][pl.ds(0, 16)] = vw

                fc[k] = jnp.where(do, fk - FLUSH, fk)
                fc[NCHUNK + k] = jnp.where(do, ok + FLUSH, ok)
            return tuple(fc)

        carry = lax.fori_loop(0, EB // 16, batch, carry)
        for j in range(16):
            pltpu.make_async_copy(wstg.at[j], degsh.at[cstg.at[j]], dsem).wait()
        return carry

    carry = (0, 0, 0, 0, 0, 0, 0, 0)
    for _blk in range(NBLOCKS):
        carry = process_block(_blk, carry)

    cv = jnp.zeros((16,), jnp.int32)
    for k in range(NCHUNK):
        fk = carry[k]
        ok = carry[NCHUNK + k]
        fb = ((fk + 255) // 256) * 256
        base = jnp.maximum(fb - 256, 0)
        for m in range(16):
            p0 = base + m * 16
            pos = p0 + i16
            keep = pos < fk
            sprd = (wid * 289 + pos * 37) & 8191
            ssrc[k][pl.ds(p0, 16)] = jnp.where(keep, ssrc[k][pl.ds(p0, 16)],
                                               sprd)
            sdst[k][pl.ds(p0, 16)] = jnp.where(keep, sdst[k][pl.ds(p0, 16)],
                                               sprd)
            sew[k][pl.ds(p0, 16)] = jnp.where(keep, sew[k][pl.ds(p0, 16)], 0.0)
        bb = pl.multiple_of((wid * NCHUNK + k) * CAP + ok, 256)
        pltpu.sync_copy(ssrc[k].at[pl.ds(0, FINAL)],
                        bsrc.at[pl.ds(bb, FINAL)])
        pltpu.sync_copy(sdst[k].at[pl.ds(0, FINAL)],
                        bdst.at[pl.ds(bb, FINAL)])
        pltpu.sync_copy(sew[k].at[pl.ds(0, FINAL)],
                        bew.at[pl.ds(bb, FINAL)])
        cv = jnp.where(i16 == k, ok + fb, cv)

    cbuf[...] = cv
    pltpu.sync_copy(cbuf, counts.at[wid])
    plsc.subcore_barrier()
    off = pl.multiple_of(s * 2560, 256)
    pltpu.sync_copy(degsh.at[pl.ds(off, 2560)],
                    degp.at[c, pl.ds(off, 2560)])


@functools.cache
def _bucket():
    return pl.kernel(
        _bucket_body,
        out_type=[
        jax.ShapeDtypeStruct((NT * NCHUNK * CAP,), jnp.int32),
        jax.ShapeDtypeStruct((NT * NCHUNK * CAP,), jnp.int32),
        jax.ShapeDtypeStruct((NT * NCHUNK * CAP,), jnp.float32),
        jax.ShapeDtypeStruct((NT, 16), jnp.int32),
        jax.ShapeDtypeStruct((2, N_PAD), jnp.float32),
    ],
        mesh=_mesh(),
        compiler_params=pltpu.CompilerParams(needs_layout_passes=False),
        scratch_types=[
        pltpu.VMEM((16, 128), jnp.int32),
        pltpu.VMEM((16, 128), jnp.int32),
        pltpu.VMEM((16, 128), jnp.float32),
        pltpu.VMEM((SCAP,), jnp.int32),
        pltpu.VMEM((SCAP,), jnp.int32),
        pltpu.VMEM((SCAP,), jnp.int32),
        pltpu.VMEM((SCAP,), jnp.int32),
        pltpu.VMEM((SCAP,), jnp.int32),
        pltpu.VMEM((SCAP,), jnp.int32),
        pltpu.VMEM((SCAP,), jnp.int32),
        pltpu.VMEM((SCAP,), jnp.int32),
        pltpu.VMEM((SCAP,), jnp.float32),
        pltpu.VMEM((SCAP,), jnp.float32),
        pltpu.VMEM((SCAP,), jnp.float32),
        pltpu.VMEM((SCAP,), jnp.float32),
        pltpu.VMEM((2560,), jnp.float32),
        pltpu.VMEM((16,), jnp.int32),
        pltpu.VMEM_SHARED((N_PAD,), jnp.float32),
        pltpu.SemaphoreType.DMA,
    ],
    )


def _spmm_body(hs, bsrc, bdst, bew, counts, agg,
               ssrc, sdst, sew, ra, rb, cbuf, acc, gA, gB, sA, sB):
    c = lax.axis_index("c")
    s = lax.axis_index("s")
    i16 = lax.iota(jnp.int32, 16)
    z16f = jnp.zeros((16,), jnp.float32)

    def _zero_ra(i, carry):
        for g in range(8):
            ra[i, pl.ds(g * 16, 16)] = z16f
        return carry

    def _scale(buf, lo):
        def srow(r, carry):
            for u in range(8):
                rr = r * 8 + u
                ewv = plsc.load_gather(
                    sew, [jnp.full((16,), 1, jnp.int32) * lo + rr])
                for g in range(8):
                    buf[rr, pl.ds(g * 16, 16)] = buf[rr, pl.ds(g * 16, 16)] * ewv
            return carry

        lax.fori_loop(0, GB // 8, srow, 0)

    def _stage(base, q):
        off = pl.multiple_of(base + q * 2048, 256)
        pltpu.sync_copy(bsrc.at[pl.ds(off, 2048)], ssrc)
        pltpu.sync_copy(bdst.at[pl.ds(off, 2048)], sdst)
        pltpu.sync_copy(bew.at[pl.ds(off, 2048)], sew)

    def _fire_gather(lo, buf, sem):
        pltpu.async_copy(hs.at[ssrc.at[pl.ds(lo, GB)]], buf, sem)

    def _wait_gather(buf, sem):
        pltpu.make_async_copy(hs.at[ssrc.at[pl.ds(0, GB)]], buf, sem).wait()

    def _fire_scatter(lo, buf, sem):
        for u in range(8):
            idx = sdst[pl.ds(lo + u * 16, 16)]
            pltpu.async_copy(buf.at[pl.ds(u * 16, 16)], acc.at[idx], sem,
                             add=True)

    def _wait_scatter(buf, sem):
        for u in range(8):
            idx = sdst[pl.ds(u * 16, 16)]
            pltpu.make_async_copy(buf.at[pl.ds(u * 16, 16)], acc.at[idx],
                                  sem).wait()

    for ci in range(2):
        k = c * 2 + ci
        plsc.subcore_barrier()
        lax.fori_loop(0, GB, _zero_ra, 0)
        for m in range(ACC_ROWS // 16 // 64):
            pltpu.sync_copy(ra.at[pl.ds(0, 64)], acc.at[pl.ds(
                pl.multiple_of(s * (ACC_ROWS // 16) + m * 64, 64), 64)])
        plsc.subcore_barrier()
        for bi in range(2):
            t = s * 2 + bi
            pltpu.sync_copy(counts.at[t], cbuf)
            cnt = cbuf[...].at[jnp.full((16,), k, jnp.int32)].get(
                mode="promise_in_bounds")[0]
            np2 = cnt // 256
            base = pl.multiple_of((t * NCHUNK + k) * CAP, 256)

            @pl.when(np2 > 0)
            def _process(k=k, np2=np2, base=base):
                _stage(base, 0)
                _fire_gather(0, ra, gA)
                _fire_gather(GB, rb, gB)

                def pair(p, carry):
                    lo = (p & 7) * 256
                    _wait_gather(ra, gA)
                    _scale(ra, lo)
                    _fire_scatter(lo, ra, sA)
                    _wait_gather(rb, gB)
                    _scale(rb, lo + GB)
                    _fire_scatter(lo + GB, rb, sB)

                    pn = p + 1

                    @pl.when(((pn & 7) == 0) & (pn < np2))
                    def _():
                        _stage(base, pn >> 3)

                    lon = (jnp.minimum(pn, np2 - 1) & 7) * 256
                    _wait_scatter(ra, sA)
                    _fire_gather(lon, ra, gA)
                    _wait_scatter(rb, sB)
                    _fire_gather(lon + GB, rb, gB)
                    return carry

                lax.fori_loop(0, np2, pair, 0)
                _wait_gather(ra, gA)
                _wait_gather(rb, gB)

        plsc.subcore_barrier()
        nr = ACC_ROWS // 16
        pltpu.sync_copy(acc.at[pl.ds(pl.multiple_of(s * nr, 64), nr)],
                        agg.at[pl.ds(pl.multiple_of(k * ACC_ROWS + s * nr, 64),
                                     nr)])


@functools.cache
def _spmm():
    return pl.kernel(
        _spmm_body,
        out_type=jax.ShapeDtypeStruct((NCHUNK * ACC_ROWS, IN_CH), jnp.float32),
        mesh=_mesh(),
        compiler_params=pltpu.CompilerParams(needs_layout_passes=False),
        scratch_types=[
        pltpu.VMEM((2048,), jnp.int32),
        pltpu.VMEM((2048,), jnp.int32),
        pltpu.VMEM((2048,), jnp.float32),
        pltpu.VMEM((GB, 128), jnp.float32),
        pltpu.VMEM((GB, 128), jnp.float32),
        pltpu.VMEM((16,), jnp.int32),
        pltpu.VMEM_SHARED((ACC_ROWS, 128), jnp.float32),
        pltpu.SemaphoreType.DMA,
        pltpu.SemaphoreType.DMA,
        pltpu.SemaphoreType.DMA,
        pltpu.SemaphoreType.DMA,
    ],
    )


def _mm1_body(x_ref, w_ref, degp_ref, hs_ref, dinv_ref):
    d = 1.0 + degp_ref[0, :] + degp_ref[1, :]
    dinv = lax.rsqrt(d)
    h = jnp.dot(x_ref[...], w_ref[...], preferred_element_type=jnp.float32)
    hs_ref[...] = h * dinv[:, None]
    dinv_ref[...] = dinv[:, None]


def _mm1(x, w, degp):
    BR = 512
    nb = pl.cdiv(N_TOT, BR)
    return pl.pallas_call(
        _mm1_body,
        grid=(nb,),
        in_specs=[
            pl.BlockSpec((BR, IN_CH), lambda i: (i, 0)),
            pl.BlockSpec((IN_CH, HID), lambda i: (0, 0)),
            pl.BlockSpec((2, BR), lambda i: (0, i)),
        ],
        out_specs=[
            pl.BlockSpec((BR, HID), lambda i: (i, 0)),
            pl.BlockSpec((BR, 1), lambda i: (i, 0)),
        ],
        out_shape=[
            jax.ShapeDtypeStruct((N_TOT, HID), jnp.float32),
            jax.ShapeDtypeStruct((N_TOT, 1), jnp.float32),
        ],
    )(x, w, degp)


def _mm2_body(agg_ref, hs_ref, dinv_ref, bc_ref, g_ref, be_ref, w_ref,
              h1_ref, hs2_ref):
    dinv = dinv_ref[...]
    pre = dinv * (agg_ref[...] + hs_ref[...]) + bc_ref[...]
    h1 = jnp.maximum(pre, 0.0)
    h1 = g_ref[...] * (h1 * _BN_SCALE) + be_ref[...]
    h1_ref[...] = h1
    hs2_ref[...] = jnp.dot(
        h1, w_ref[...], preferred_element_type=jnp.float32) * dinv


def _mm2(agg, hs, dinv, bc, g, be, w):
    BR = 512
    nb = pl.cdiv(N_TOT, BR)
    return pl.pallas_call(
        _mm2_body,
        grid=(nb,),
        in_specs=[
            pl.BlockSpec((BR, HID), lambda i: (i, 0)),
            pl.BlockSpec((BR, HID), lambda i: (i, 0)),
            pl.BlockSpec((BR, 1), lambda i: (i, 0)),
            pl.BlockSpec((1, HID), lambda i: (0, 0)),
            pl.BlockSpec((1, HID), lambda i: (0, 0)),
            pl.BlockSpec((1, HID), lambda i: (0, 0)),
            pl.BlockSpec((HID, HID), lambda i: (0, 0)),
        ],
        out_specs=[
            pl.BlockSpec((BR, HID), lambda i: (i, 0)),
            pl.BlockSpec((BR, HID), lambda i: (i, 0)),
        ],
        out_shape=[
            jax.ShapeDtypeStruct((N_TOT, HID), jnp.float32),
            jax.ShapeDtypeStruct((N_TOT, HID), jnp.float32),
        ],
    )(agg, hs, dinv, bc, g, be, w)


def _sigm(v):
    return jax.nn.sigmoid(v)


def _lstm_head_body(h1_ref, agg2_ref, hs2_ref, dinv_ref, bc2_ref, g2_ref,
                    be2_ref, wih1_ref, whh1_ref, b1_ref, wih2_ref, whh2_ref,
                    b2_ref, x_ref, wl1_ref, bl1_ref, wl2_ref, bl2_ref, o_ref):
    BR = h1_ref.shape[1]
    h1 = h1_ref[...]
    dinv = dinv_ref[...]
    h2 = dinv * (agg2_ref[...] + hs2_ref[...]) + bc2_ref[...]
    h2 = jnp.maximum(h2, 0.0)
    h2 = g2_ref[...] * (h2 * _BN_SCALE) + be2_ref[...]
    bf = jnp.bfloat16
    wih1 = wih1_ref[...].astype(bf)
    whh1 = whh1_ref[...].astype(bf)
    b1 = b1_ref[...]
    wih2 = wih2_ref[...].astype(bf)
    whh2 = whh2_ref[...].astype(bf)
    b2 = b2_ref[...]
    hA = jnp.zeros((BR, HID), jnp.float32)
    cA = jnp.zeros((BR, HID), jnp.float32)
    hB = jnp.zeros((BR, HID), jnp.float32)
    cB = jnp.zeros((BR, HID), jnp.float32)
    for t in range(WINDOW):
        xt = jnp.concatenate([h1[t], h2[t]], axis=1).astype(bf)
        gates = (jnp.dot(xt, wih1.T, preferred_element_type=jnp.float32)
                 + jnp.dot(hA.astype(bf), whh1.T,
                           preferred_element_type=jnp.float32)
                 + b1)
        ig, fg, gg, og = jnp.split(gates, 4, axis=1)
        cA = _sigm(fg) * cA + _sigm(ig) * jnp.tanh(gg)
        hA = _sigm(og) * jnp.tanh(cA)
        gates = (jnp.dot(hA.astype(bf), wih2.T,
                         preferred_element_type=jnp.float32)
                 + jnp.dot(hB.astype(bf), whh2.T,
                           preferred_element_type=jnp.float32)
                 + b2)
        ig, fg, gg, og = jnp.split(gates, 4, axis=1)
        cB = _sigm(fg) * cB + _sigm(ig) * jnp.tanh(gg)
        hB = _sigm(og) * jnp.tanh(cB)
    x4 = x_ref[...]
    S = jnp.concatenate(
        [x4[0], x4[1][:, IN_CH - 1:], x4[2][:, IN_CH - 1:],
         x4[3][:, IN_CH - 1:]], axis=1)
    hcat = jnp.maximum(jnp.concatenate([hA, hB, S], axis=1), 0.0)
    z = jnp.maximum(
        jnp.dot(hcat, wl1_ref[...].T, preferred_element_type=jnp.float32)
        + bl1_ref[...], 0.0)
    o_ref[...] = (jnp.dot(z, wl2_ref[...].T,
                          preferred_element_type=jnp.float32) + bl2_ref[...])


def _lstm_head(h1r, agg2r, hs2r, dinvr, bc2, g2, be2, Wih1, Whh1, b1,
               Wih2, Whh2, b2, xr, Wl1, bl1, Wl2, bl2):
    BR = 512
    nb = pl.cdiv(NUM_NODES, BR)
    d1 = 2 * HID + IN_CH + WINDOW - 1
    full = lambda shape: pl.BlockSpec(shape, lambda i: tuple(0 for _ in shape))
    return pl.pallas_call(
        _lstm_head_body,
        grid=(nb,),
        in_specs=[
            pl.BlockSpec((WINDOW, BR, HID), lambda i: (0, i, 0)),
            pl.BlockSpec((WINDOW, BR, HID), lambda i: (0, i, 0)),
            pl.BlockSpec((WINDOW, BR, HID), lambda i: (0, i, 0)),
            pl.BlockSpec((WINDOW, BR, 1), lambda i: (0, i, 0)),
            full((1, HID)),
            full((1, HID)),
            full((1, HID)),
            full((4 * HID, 2 * HID)),
            full((4 * HID, HID)),
            full((1, 4 * HID)),
            full((4 * HID, HID)),
            full((4 * HID, HID)),
            full((1, 4 * HID)),
            pl.BlockSpec((WINDOW, BR, IN_CH), lambda i: (0, i, 0)),
            full((HID, d1)),
            full((1, HID)),
            full((D_D, HID)),
            full((1, D_D)),
        ],
        out_specs=pl.BlockSpec((BR, D_D), lambda i: (i, 0)),
        out_shape=jax.ShapeDtypeStruct((NUM_NODES, D_D), jnp.float32),
    )(h1r, agg2r, hs2r, dinvr, bc2, g2, be2, Wih1, Whh1, b1, Wih2, Whh2, b2,
      xr, Wl1, bl1, Wl2, bl2)


def kernel(x, edge_index, edge_weight, Wc1, bc1, Wc2, bc2, g1, be1, g2, be2,
           Wih1, Whh1, bih1, bhh1, Wih2, Whh2, bih2, bhh2, Wl1, bl1, Wl2, bl2):
    e = edge_weight.shape[0]
    pad = EPAD - e
    spread = (jnp.arange(pad, dtype=edge_index.dtype) * N_TOT) // pad
    row2 = jnp.concatenate([edge_index[0], spread]).reshape(-1, 128)
    col2 = jnp.concatenate([edge_index[1], spread]).reshape(-1, 128)
    ew2 = jnp.concatenate(
        [edge_weight, jnp.zeros((pad,), edge_weight.dtype)]).reshape(-1, 128)

    bsrc, bdst, bew, counts, degp = _bucket()(row2, col2, ew2)

    unpad = lambda a: a.reshape(NCHUNK, ACC_ROWS, IN_CH)[:, :NUM_NODES, :]
    hs1, dinv = _mm1(x, Wc1, degp)
    agg1 = unpad(_spmm()(hs1, bsrc, bdst, bew, counts)).reshape(N_TOT, IN_CH)
    h1, hs2 = _mm2(agg1, hs1, dinv, bc1.reshape(1, -1), g1.reshape(1, -1),
                   be1.reshape(1, -1), Wc2)
    agg2 = unpad(_spmm()(hs2, bsrc, bdst, bew, counts)).reshape(N_TOT, IN_CH)

    r4 = lambda a: a.reshape(WINDOW, NUM_NODES, -1)
    out = _lstm_head(
        r4(h1), r4(agg2), r4(hs2), r4(dinv), bc2.reshape(1, -1),
        g2.reshape(1, -1), be2.reshape(1, -1), Wih1, Whh1,
        (bih1 + bhh1).reshape(1, -1), Wih2, Whh2,
        (bih2 + bhh2).reshape(1, -1), r4(x), Wl1, bl1.reshape(1, -1),
        Wl2, bl2.reshape(1, -1))
    return out
